# Initial kernel scaffold; baseline (speedup 1.0000x reference)
import functools

import jax
import jax.numpy as jnp
from jax import lax
from jax.experimental import pallas as pl
from jax.experimental.pallas import tpu as pltpu
import jax.experimental.pallas.tpu_sc as plsc

N = 10000
D = 256
E = 160000
STEP = 0.5

NCORE = 2
NSUB = 16
N2 = N // NCORE
N_PAD = 10240
CHUNK = 96
GROUPS = CHUNK // 16
CPT = 106
EPT = CPT * CHUNK
E_PAD = EPT * NSUB
TROWS = 320
PCH = 32


def _tanh16(z):
    az = jnp.abs(z)
    e = jnp.exp(-2.0 * az)
    p = (1.0 - e) / (1.0 + e)
    return jnp.where(z < 0, -p, p)


def _rsqrt16(u):
    g = plsc.bitcast(jnp.int32(0x5F3759DF) - (plsc.bitcast(u, jnp.int32) >> 1),
                     jnp.float32)
    for _ in range(3):
        g = g * (1.5 - 0.5 * u * g * g)
    return g


def _dense(x, w_lin, b_lin2, wsp):
    BM = 512
    grid = (pl.cdiv(N, BM),)

    def body(x_ref, wl_ref, bl_ref, ws_ref, y_ref, ab_ref):
        xb = x_ref[...]
        y = lax.dot_general(xb, wl_ref[...], (((1,), (1,)), ((), ())),
                            preferred_element_type=jnp.float32)
        y_ref[...] = y + bl_ref[...]
        ab_ref[...] = jnp.dot(xb, ws_ref[...], preferred_element_type=jnp.float32)

    return pl.pallas_call(
        body,
        grid=grid,
        in_specs=[
            pl.BlockSpec((BM, D), lambda i: (i, 0)),
            pl.BlockSpec((D, D), lambda i: (0, 0)),
            pl.BlockSpec((1, D), lambda i: (0, 0)),
            pl.BlockSpec((D, 128), lambda i: (0, 0)),
        ],
        out_specs=[
            pl.BlockSpec((BM, D), lambda i: (i, 0)),
            pl.BlockSpec((BM, 128), lambda i: (i, 0)),
        ],
        out_shape=[
            jax.ShapeDtypeStruct((N, D), jnp.float32),
            jax.ShapeDtypeStruct((N, 128), jnp.float32),
        ],
    )(x, w_lin, b_lin2, wsp)


def _sc_body(x_hbm, y_hbm, ab_hbm, row_hbm, col_hbm, out_hbm,
             ab_t, dsi_t, rowb, colb, ndb, wb, widxb, pay, rows,
             dwbuf, dsich, dscch, bx, by, bs,
             acc, dwide, dsi_s, dsc_s, nd_s, sem):
    c = lax.axis_index("c")
    s = lax.axis_index("s")
    iota16 = lax.iota(jnp.int32, 16)
    zero16i = jnp.zeros((16,), jnp.int32)
    zf = jnp.zeros((16,), jnp.float32)

    pltpu.sync_copy(ab_hbm, ab_t.at[pl.ds(0, 2 * N)])
    for i in range(2 * (N_PAD - N) // 16):
        ab_t[pl.ds(2 * N + i * 16, 16)] = zf
    for r in range(CHUNK):
        pay[r, :] = zf
    for r in range(PCH):
        for j in range(D // 16):
            bs[r, pl.ds(j * 16, 16)] = zf
    tb = s * TROWS
    for k in range(TROWS // PCH):
        pltpu.sync_copy(bs, acc.at[pl.ds(tb + k * PCH, PCH), :])
    db = s * (N_PAD // NSUB)
    for k in range(6):
        pltpu.sync_copy(pay, dwide.at[pl.ds(db + k * CHUNK, CHUNK), :])
    pltpu.sync_copy(pay.at[pl.ds(0, 64), :], dwide.at[pl.ds(db + 576, 64), :])
    plsc.subcore_barrier()

    ebase = s * EPT

    def p1_body(i, carry):
        off = ebase + i * CHUNK
        pltpu.sync_copy(row_hbm.at[pl.ds(off, CHUNK)], rowb)
        pltpu.sync_copy(col_hbm.at[pl.ds(off, CHUNK)], colb)
        for g in range(GROUPS):
            r16 = rowb[pl.ds(g * 16, 16)]
            c16 = colb[pl.ds(g * 16, 16)]
            ar = plsc.load_gather(ab_t, [r16 * 2])
            bc = plsc.load_gather(ab_t, [c16 * 2 + 1])
            ac = plsc.load_gather(ab_t, [c16 * 2])
            br = plsc.load_gather(ab_t, [r16 * 2 + 1])
            m = _tanh16(ar + bc)
            rm = _tanh16(ac + br)
            plsc.store_scatter(pay, [iota16 + g * 16, zero16i], m * m)
            ndb[pl.ds(g * 16, 16)] = -(m * rm)
        pltpu.sync_copy(pay, dwide.at[rowb], add=True)
        pltpu.sync_copy(ndb, nd_s.at[pl.ds(off, CHUNK)])
        return carry

    lax.fori_loop(0, CPT, p1_body, 0)
    plsc.subcore_barrier()

    pltpu.sync_copy(dwide.at[pl.ds(db, N_PAD // NSUB), :], dwbuf)
    for i in range(N_PAD // NSUB // 16):
        v = plsc.load_gather(dwbuf, [iota16 + i * 16, zero16i])
        u = v + 1.0
        dsich[pl.ds(i * 16, 16)] = _rsqrt16(u)
        dscch[pl.ds(i * 16, 16)] = v / u
    pltpu.sync_copy(dsich, dsi_s.at[pl.ds(db, N_PAD // NSUB)])
    pltpu.sync_copy(dscch, dsc_s.at[pl.ds(db, N_PAD // NSUB)])
    plsc.subcore_barrier()
    pltpu.sync_copy(dsi_s, dsi_t)

    nbase = c * N2

    def p3_body(i, carry):
        off = ebase + i * CHUNK
        pltpu.sync_copy(col_hbm.at[pl.ds(off, CHUNK)], colb)
        cp = pltpu.async_copy(y_hbm.at[colb], rows, sem)
        pltpu.sync_copy(row_hbm.at[pl.ds(off, CHUNK)], rowb)
        pltpu.sync_copy(nd_s.at[pl.ds(off, CHUNK)], ndb)
        for g in range(GROUPS):
            r16 = rowb[pl.ds(g * 16, 16)]
            c16 = colb[pl.ds(g * 16, 16)]
            nd16 = ndb[pl.ds(g * 16, 16)]
            w = plsc.load_gather(dsi_t, [r16]) * nd16 * plsc.load_gather(dsi_t, [c16])
            loc = r16 - nbase
            inb = (loc >= 0) & (loc < N2)
            wb[pl.ds(g * 16, 16)] = jnp.where(inb, w, 0.0)
            widxb[pl.ds(g * 16, 16)] = jnp.where(inb, loc, 0)
        cp.wait()

        def scale_body(e, carry2):
            ws = wb[e]
            for j in range(D // 16):
                sl = pl.ds(j * 16, 16)
                rows[e, sl] = rows[e, sl] * ws
            return carry2

        lax.fori_loop(0, CHUNK, scale_body, 0)
        pltpu.sync_copy(rows, acc.at[widxb], add=True)
        return carry

    lax.fori_loop(0, CPT, p3_body, 0)
    plsc.subcore_barrier()

    def combine(gb, lb, nr):
        pltpu.sync_copy(x_hbm.at[pl.ds(gb, nr), :], bx.at[pl.ds(0, nr), :])
        pltpu.sync_copy(y_hbm.at[pl.ds(gb, nr), :], by.at[pl.ds(0, nr), :])
        pltpu.sync_copy(acc.at[pl.ds(lb, nr), :], bs.at[pl.ds(0, nr), :])
        pltpu.sync_copy(dsc_s.at[pl.ds(gb, nr)], dscch.at[pl.ds(0, nr)])

        def rbody(r, carry2):
            dv = dscch[r]
            for j in range(D // 16):
                sl = pl.ds(j * 16, 16)
                bx[r, sl] = bx[r, sl] - STEP * (dv * by[r, sl] + bs[r, sl])
            return carry2

        lax.fori_loop(0, nr, rbody, 0)
        pltpu.sync_copy(bx.at[pl.ds(0, nr), :], out_hbm.at[pl.ds(gb, nr), :])

    def p4_body(k, carry):
        lb = s * TROWS + k * PCH
        gb = nbase + lb

        @pl.when(lb + PCH <= N2)
        def _():
            combine(gb, lb, PCH)

        @pl.when((lb < N2) & (lb + PCH > N2))
        def _():
            combine(gb, lb, 8)

        return carry

    lax.fori_loop(0, TROWS // PCH, p4_body, 0)


def _sc_stage(x, y, ab_flat, row_p, col_p):
    mesh = plsc.VectorSubcoreMesh(core_axis_name="c", subcore_axis_name="s",
                                  num_cores=NCORE, num_subcores=NSUB)
    f = functools.partial(
        pl.kernel,
        mesh=mesh,
        out_type=jax.ShapeDtypeStruct((N, D), jnp.float32),
        scratch_types=[
            pltpu.VMEM((2 * N_PAD,), jnp.float32),
            pltpu.VMEM((N_PAD,), jnp.float32),
            pltpu.VMEM((CHUNK,), jnp.int32),
            pltpu.VMEM((CHUNK,), jnp.int32),
            pltpu.VMEM((CHUNK,), jnp.float32),
            pltpu.VMEM((CHUNK,), jnp.float32),
            pltpu.VMEM((CHUNK,), jnp.int32),
            pltpu.VMEM((CHUNK, 16), jnp.float32),
            pltpu.VMEM((CHUNK, D), jnp.float32),
            pltpu.VMEM((N_PAD // NSUB, 16), jnp.float32),
            pltpu.VMEM((N_PAD // NSUB,), jnp.float32),
            pltpu.VMEM((N_PAD // NSUB,), jnp.float32),
            pltpu.VMEM((PCH, D), jnp.float32),
            pltpu.VMEM((PCH, D), jnp.float32),
            pltpu.VMEM((PCH, D), jnp.float32),
            pltpu.VMEM_SHARED((NSUB * TROWS, D), jnp.float32),
            pltpu.VMEM_SHARED((N_PAD, 16), jnp.float32),
            pltpu.VMEM_SHARED((N_PAD,), jnp.float32),
            pltpu.VMEM_SHARED((N_PAD,), jnp.float32),
            pltpu.VMEM_SHARED((E_PAD,), jnp.float32),
            pltpu.SemaphoreType.DMA,
        ],
    )(_sc_body)
    return f(x, y, ab_flat, row_p, col_p)


@jax.jit
def kernel(x, edge_index, W_lin, b_lin, W_sheaf):
    wsp = jnp.zeros((D, 128), jnp.float32)
    wsp = wsp.at[:, 0].set(W_sheaf[:D, 0]).at[:, 1].set(W_sheaf[D:, 0])
    y, abp = _dense(x, W_lin, b_lin.reshape(1, D), wsp)
    ab_flat = abp[:, :2].reshape(-1)
    pad = E_PAD - E
    row_p = jnp.concatenate([edge_index[0], jnp.full((pad,), N, jnp.int32)])
    col_p = jnp.concatenate([edge_index[1], jnp.zeros((pad,), jnp.int32)])
    return _sc_stage(x, y, ab_flat, row_p, col_p)


# speedup vs baseline: 4.2497x; 4.2497x over previous
import functools

import jax
import jax.numpy as jnp
from jax import lax
from jax.experimental import pallas as pl
from jax.experimental.pallas import tpu as pltpu
import jax.experimental.pallas.tpu_sc as plsc

N = 10000
D = 256
E = 160000
STEP = 0.5

NCORE = 2
NSUB = 16
N2 = N // NCORE
N_PAD = 10240
CHUNK = 48
GROUPS = CHUNK // 16
CPT = 209
EPT = CPT * CHUNK
E_PAD = EPT * NSUB
TROWS = 320
PCH = 16
NTAIL = N2 - (NSUB - 1) * TROWS - 12 * PCH


def _tanh16(z):
    az = jnp.abs(z)
    e = jnp.exp(-2.0 * az)
    p = (1.0 - e) / (1.0 + e)
    return jnp.where(z < 0, -p, p)


def _rsqrt16(u):
    g = plsc.bitcast(jnp.int32(0x5F3759DF) - (plsc.bitcast(u, jnp.int32) >> 1),
                     jnp.float32)
    for _ in range(3):
        g = g * (1.5 - 0.5 * u * g * g)
    return g


def _dense(x, w_lin, b_lin2, wsp):
    BM = 512
    grid = (pl.cdiv(N, BM),)

    def body(x_ref, wl_ref, bl_ref, ws_ref, y_ref, ab_ref):
        xb = x_ref[...]
        y = lax.dot_general(xb, wl_ref[...], (((1,), (1,)), ((), ())),
                            preferred_element_type=jnp.float32)
        y_ref[...] = y + bl_ref[...]
        ab_ref[...] = jnp.dot(xb, ws_ref[...], preferred_element_type=jnp.float32)

    return pl.pallas_call(
        body,
        grid=grid,
        in_specs=[
            pl.BlockSpec((BM, D), lambda i: (i, 0)),
            pl.BlockSpec((D, D), lambda i: (0, 0)),
            pl.BlockSpec((1, D), lambda i: (0, 0)),
            pl.BlockSpec((D, 128), lambda i: (0, 0)),
        ],
        out_specs=[
            pl.BlockSpec((BM, D), lambda i: (i, 0)),
            pl.BlockSpec((BM, 128), lambda i: (i, 0)),
        ],
        out_shape=[
            jax.ShapeDtypeStruct((N, D), jnp.float32),
            jax.ShapeDtypeStruct((N, 128), jnp.float32),
        ],
    )(x, w_lin, b_lin2, wsp)


def _sc_body(x_hbm, y_hbm, ab_hbm, row_hbm, col_hbm, out_hbm,
             tab, rowb, colb, ndb, wb, gidx, sidx, m2b, rows,
             dgch, dsich, dscch,
             acc, dwide, dsi_s, nd_s, sem):
    c = lax.axis_index("c")
    s = lax.axis_index("s")
    zf = jnp.zeros((16,), jnp.float32)

    pltpu.sync_copy(ab_hbm, tab.at[pl.ds(0, 2 * N)])

    def z_tab(i, carry):
        tab[pl.ds(2 * N + i * 16, 16)] = zf
        return carry

    lax.fori_loop(0, 2 * (N_PAD - N) // 16, z_tab, 0)

    def z_rows(r, carry):
        for j in range(128 // 16):
            rows[64 + r, pl.ds(j * 16, 16)] = zf
        return carry

    lax.fori_loop(0, 2 * PCH, z_rows, 0)
    tb = 2 * s * TROWS

    def z_acc(k, carry):
        pltpu.sync_copy(rows.at[pl.ds(64, 2 * PCH), :],
                        acc.at[pl.ds(tb + k * 2 * PCH, 2 * PCH), :])
        return carry

    lax.fori_loop(0, TROWS // PCH, z_acc, 0)
    db = s * (N_PAD // NSUB)

    def z_dsich(i, carry):
        dsich[pl.ds(i * 16, 16)] = zf
        return carry

    lax.fori_loop(0, N_PAD // NSUB // 16, z_dsich, 0)
    pltpu.sync_copy(dsich, dwide.at[pl.ds(db, N_PAD // NSUB)])
    plsc.subcore_barrier()

    ebase = s * EPT

    def p1_body(i, carry):
        off = ebase + i * CHUNK
        pltpu.sync_copy(row_hbm.at[pl.ds(off, CHUNK)], rowb)
        pltpu.sync_copy(col_hbm.at[pl.ds(off, CHUNK)], colb)
        for g in range(GROUPS):
            r16 = rowb[pl.ds(g * 16, 16)]
            c16 = colb[pl.ds(g * 16, 16)]
            ar = plsc.load_gather(tab, [r16 * 2])
            bc = plsc.load_gather(tab, [c16 * 2 + 1])
            ac = plsc.load_gather(tab, [c16 * 2])
            br = plsc.load_gather(tab, [r16 * 2 + 1])
            m = _tanh16(ar + bc)
            rm = _tanh16(ac + br)
            m2b[pl.ds(g * 16, 16)] = m * m
            ndb[pl.ds(g * 16, 16)] = -(m * rm)
        pltpu.sync_copy(m2b, dwide.at[rowb], add=True)
        pltpu.sync_copy(ndb, nd_s.at[pl.ds(off, CHUNK)])
        return carry

    lax.fori_loop(0, CPT, p1_body, 0)
    plsc.subcore_barrier()

    pltpu.sync_copy(dwide.at[pl.ds(db, N_PAD // NSUB)], dgch)

    def p2_body(i, carry):
        v = dgch[pl.ds(i * 16, 16)]
        u = v + 1.0
        dsich[pl.ds(i * 16, 16)] = _rsqrt16(u)
        dscch[pl.ds(i * 16, 16)] = v / u
        return carry

    lax.fori_loop(0, N_PAD // NSUB // 16, p2_body, 0)
    pltpu.sync_copy(dsich, dsi_s.at[pl.ds(db, N_PAD // NSUB)])
    pltpu.sync_copy(dscch, dwide.at[pl.ds(db, N_PAD // NSUB)])
    plsc.subcore_barrier()
    pltpu.sync_copy(dsi_s, tab.at[pl.ds(0, N_PAD)])

    nbase = c * N2

    iota16 = lax.iota(jnp.int32, 16)

    def p3_body(i, carry):
        off = ebase + i * CHUNK
        pltpu.sync_copy(col_hbm.at[pl.ds(off, CHUNK)], colb)
        pltpu.sync_copy(row_hbm.at[pl.ds(off, CHUNK)], rowb)
        pltpu.sync_copy(nd_s.at[pl.ds(off, CHUNK)], ndb)
        for g in range(GROUPS):
            c16 = colb[pl.ds(g * 16, 16)]
            even = iota16 * 2 + g * 32
            plsc.store_scatter(gidx, [even], c16 * 2)
            plsc.store_scatter(gidx, [even + 1], c16 * 2 + 1)
        cp = pltpu.async_copy(y_hbm.at[gidx], rows.at[pl.ds(0, 2 * CHUNK), :], sem)
        for g in range(GROUPS):
            r16 = rowb[pl.ds(g * 16, 16)]
            c16 = colb[pl.ds(g * 16, 16)]
            nd16 = ndb[pl.ds(g * 16, 16)]
            w = plsc.load_gather(tab, [r16]) * nd16 * plsc.load_gather(tab, [c16])
            loc = r16 - nbase
            inb = (loc >= 0) & (loc < N2)
            wb[pl.ds(g * 16, 16)] = jnp.where(inb, w, 0.0)
            loc = jnp.where(inb, loc, 0)
            even = iota16 * 2 + g * 32
            plsc.store_scatter(sidx, [even], loc * 2)
            plsc.store_scatter(sidx, [even + 1], loc * 2 + 1)
        cp.wait()

        def scale_body(e, carry2):
            ws = plsc.load_gather(wb, [jnp.full((16,), e, jnp.int32)])
            for j in range(128 // 16):
                sl = pl.ds(j * 16, 16)
                rows[2 * e, sl] = rows[2 * e, sl] * ws
                rows[2 * e + 1, sl] = rows[2 * e + 1, sl] * ws
            return carry2

        lax.fori_loop(0, CHUNK, scale_body, 0)
        pltpu.sync_copy(rows.at[pl.ds(0, 2 * CHUNK), :], acc.at[sidx], add=True)
        return carry

    lax.fori_loop(0, CPT, p3_body, 0)
    plsc.subcore_barrier()

    def combine(gb, lb, nr):
        pltpu.sync_copy(x_hbm.at[pl.ds(2 * gb, 2 * nr), :], rows.at[pl.ds(0, 2 * nr), :])
        pltpu.sync_copy(y_hbm.at[pl.ds(2 * gb, 2 * nr), :], rows.at[pl.ds(32, 2 * nr), :])
        pltpu.sync_copy(acc.at[pl.ds(2 * lb, 2 * nr), :], rows.at[pl.ds(64, 2 * nr), :])
        pltpu.sync_copy(dwide.at[pl.ds(gb, nr)], dscch.at[pl.ds(0, nr)])

        def rbody(lane, carry2):
            dv = plsc.load_gather(dscch, [jnp.full((16,), lane, jnp.int32)])
            for h in range(2):
                r = 2 * lane + h
                for j in range(128 // 16):
                    sl = pl.ds(j * 16, 16)
                    rows[r, sl] = rows[r, sl] - STEP * (
                        dv * rows[32 + r, sl] + rows[64 + r, sl])
            return carry2

        lax.fori_loop(0, nr, rbody, 0)
        pltpu.sync_copy(rows.at[pl.ds(0, 2 * nr), :], out_hbm.at[pl.ds(2 * gb, 2 * nr), :])

    nfull = jnp.where(s == NSUB - 1, (N2 - (NSUB - 1) * TROWS) // PCH,
                      TROWS // PCH)

    def p4_body(k, carry):
        lb = s * TROWS + k * PCH
        combine(nbase + lb, lb, PCH)
        return carry

    lax.fori_loop(0, nfull, p4_body, 0)

    @pl.when(s == NSUB - 1)
    def _():
        lb = (NSUB - 1) * TROWS + ((N2 - (NSUB - 1) * TROWS) // PCH) * PCH
        combine(nbase + lb, lb, NTAIL)


def _sc_stage(x, y, ab_flat, row_p, col_p):
    mesh = plsc.VectorSubcoreMesh(core_axis_name="c", subcore_axis_name="s",
                                  num_cores=NCORE, num_subcores=NSUB)
    f = functools.partial(
        pl.kernel,
        mesh=mesh,
        compiler_params=pltpu.CompilerParams(needs_layout_passes=False),
        out_type=jax.ShapeDtypeStruct((2 * N, 128), jnp.float32),
        scratch_types=[
            pltpu.VMEM((2 * N_PAD,), jnp.float32),
            pltpu.VMEM((CHUNK,), jnp.int32),
            pltpu.VMEM((CHUNK,), jnp.int32),
            pltpu.VMEM((CHUNK,), jnp.float32),
            pltpu.VMEM((CHUNK,), jnp.float32),
            pltpu.VMEM((2 * CHUNK,), jnp.int32),
            pltpu.VMEM((2 * CHUNK,), jnp.int32),
            pltpu.VMEM((CHUNK,), jnp.float32),
            pltpu.VMEM((2 * CHUNK, 128), jnp.float32),
            pltpu.VMEM((N_PAD // NSUB,), jnp.float32),
            pltpu.VMEM((N_PAD // NSUB,), jnp.float32),
            pltpu.VMEM((N_PAD // NSUB,), jnp.float32),
            pltpu.VMEM_SHARED((2 * NSUB * TROWS, 128), jnp.float32),
            pltpu.VMEM_SHARED((N_PAD,), jnp.float32),
            pltpu.VMEM_SHARED((N_PAD,), jnp.float32),
            pltpu.VMEM_SHARED((E_PAD,), jnp.float32),
            pltpu.SemaphoreType.DMA,
        ],
    )(_sc_body)
    x2 = x.reshape(2 * N, 128)
    y2 = y.reshape(2 * N, 128)
    out2 = f(x2, y2, ab_flat, row_p, col_p)
    return out2.reshape(N, D)


@jax.jit
def kernel(x, edge_index, W_lin, b_lin, W_sheaf):
    wsp = jnp.zeros((D, 128), jnp.float32)
    wsp = wsp.at[:, 0].set(W_sheaf[:D, 0]).at[:, 1].set(W_sheaf[D:, 0])
    y, abp = _dense(x, W_lin, b_lin.reshape(1, D), wsp)
    ab_flat = abp[:, :2].reshape(-1)
    pad = E_PAD - E
    row_p = jnp.concatenate([edge_index[0], jnp.full((pad,), N, jnp.int32)])
    col_p = jnp.concatenate([edge_index[1], jnp.zeros((pad,), jnp.int32)])
    return _sc_stage(x, y, ab_flat, row_p, col_p)


# speedup vs baseline: 5.6127x; 1.3207x over previous
import functools

import jax
import jax.numpy as jnp
from jax import lax
from jax.experimental import pallas as pl
from jax.experimental.pallas import tpu as pltpu
import jax.experimental.pallas.tpu_sc as plsc

N = 10000
D = 256
E = 160000
STEP = 0.5

NCORE = 2
NSUB = 16
N2 = N // NCORE
N_PAD = 10240
CHUNK = 64
GROUPS = CHUNK // 16
SUP = 4
SUPER = SUP * CHUNK
SPT = 40
CPT = SPT * SUP
EPT = CPT * CHUNK
E_PAD = EPT * NSUB
TROWS = 320
PCH = 16
NTAIL = 8

MASK_HI = -65536


def _tanh16(z):
    az = jnp.abs(z)
    e = jnp.exp(-2.0 * az)
    p = (1.0 - e) / (1.0 + e)
    return jnp.where(z < 0, -p, p)


def _rsqrt16(u):
    g = plsc.bitcast(jnp.int32(0x5F3759DF) - (plsc.bitcast(u, jnp.int32) >> 1),
                     jnp.float32)
    for _ in range(3):
        g = g * (1.5 - 0.5 * u * g * g)
    return g


def _unpack_ab(v):
    a = plsc.bitcast(v & jnp.int32(MASK_HI), jnp.float32)
    b = plsc.bitcast(v << 16, jnp.float32)
    return a, b


def _dense(x, w_lin, b_lin2, wsp):
    BM = 512
    grid = (pl.cdiv(N, BM),)

    def body(x_ref, wl_ref, bl_ref, ws_ref, y_ref, ab_ref):
        xb = x_ref[...]
        y = lax.dot_general(xb, wl_ref[...], (((1,), (1,)), ((), ())),
                            preferred_element_type=jnp.float32)
        y_ref[...] = y + bl_ref[...]
        ab_ref[...] = jnp.dot(xb, ws_ref[...], preferred_element_type=jnp.float32)

    return pl.pallas_call(
        body,
        grid=grid,
        in_specs=[
            pl.BlockSpec((BM, D), lambda i: (i, 0)),
            pl.BlockSpec((D, D), lambda i: (0, 0)),
            pl.BlockSpec((1, D), lambda i: (0, 0)),
            pl.BlockSpec((D, 128), lambda i: (0, 0)),
        ],
        out_specs=[
            pl.BlockSpec((BM, D), lambda i: (i, 0)),
            pl.BlockSpec((BM, 128), lambda i: (i, 0)),
        ],
        out_shape=[
            jax.ShapeDtypeStruct((N, D), jnp.float32),
            jax.ShapeDtypeStruct((N, 128), jnp.float32),
        ],
    )(x, w_lin, b_lin2, wsp)


def _sc_body(x_hbm, y_hbm, ab_hbm, row_hbm, col_hbm,
             out_hbm, dsi_hbm, dsc_hbm,
             tab, rowsb, colsb, rows, gidx, sidx, ridx, cidx,
             dsir, dsic, wb, m2b, rowb64,
             dgch, dsich, dscch, acc, dwide, semr, sems):
    c = lax.axis_index("c")
    s = lax.axis_index("s")
    iota16 = lax.iota(jnp.int32, 16)
    zf = jnp.zeros((16,), jnp.float32)
    zi = jnp.zeros((16,), jnp.int32)

    pltpu.sync_copy(ab_hbm, tab.at[pl.ds(0, N)])

    def z_tab(i, carry):
        tab[pl.ds(N + i * 16, 16)] = zi
        return carry

    lax.fori_loop(0, (N_PAD - N) // 16, z_tab, 0)

    def z_rows(r, carry):
        for j in range(128 // 16):
            rows[r, pl.ds(j * 16, 16)] = zf
        return carry

    lax.fori_loop(0, 2 * PCH, z_rows, 0)
    tb = 2 * s * TROWS

    def z_acc(k, carry):
        pltpu.sync_copy(rows.at[pl.ds(0, 2 * PCH), :],
                        acc.at[pl.ds(tb + k * 2 * PCH, 2 * PCH), :])
        return carry

    lax.fori_loop(0, TROWS // PCH, z_acc, 0)
    db = s * (N_PAD // NSUB)

    def z_dsich(i, carry):
        dsich[pl.ds(i * 16, 16)] = zf
        return carry

    lax.fori_loop(0, N_PAD // NSUB // 16, z_dsich, 0)
    pltpu.sync_copy(dsich, dwide.at[pl.ds(db, N_PAD // NSUB)])
    plsc.subcore_barrier()

    ebase = s * EPT

    def p1_super(s2, carry):
        soff = ebase + s2 * SUPER
        pltpu.sync_copy(row_hbm.at[pl.ds(soff, SUPER)], rowsb.at[0])
        pltpu.sync_copy(col_hbm.at[pl.ds(soff, SUPER)], colsb.at[0])

        def p1_sub(k, carry2):
            for g in range(GROUPS):
                sl16 = pl.ds(k * CHUNK + g * 16, 16)
                r16 = rowsb[0, sl16]
                c16 = colsb[0, sl16]
                ar, br = _unpack_ab(plsc.load_gather(tab, [r16]))
                ac, bc = _unpack_ab(plsc.load_gather(tab, [c16]))
                m = _tanh16(ar + bc)
                m2b[pl.ds(g * 16, 16)] = m * m
                rowb64[pl.ds(g * 16, 16)] = r16
            pltpu.sync_copy(m2b, dwide.at[rowb64], add=True)
            return carry2

        lax.fori_loop(0, SUP, p1_sub, 0)
        return carry

    lax.fori_loop(0, SPT, p1_super, 0)
    plsc.subcore_barrier()

    pltpu.sync_copy(dwide.at[pl.ds(db, N_PAD // NSUB)], dgch)

    def p2_body(i, carry):
        v = dgch[pl.ds(i * 16, 16)]
        u = v + 1.0
        dsich[pl.ds(i * 16, 16)] = _rsqrt16(u)
        dscch[pl.ds(i * 16, 16)] = v / u
        return carry

    lax.fori_loop(0, N_PAD // NSUB // 16, p2_body, 0)
    pltpu.sync_copy(dsich, dsi_hbm.at[pl.ds(db, N_PAD // NSUB)])
    pltpu.sync_copy(dscch, dsc_hbm.at[pl.ds(db, N_PAD // NSUB)])
    plsc.subcore_barrier()

    nbase = c * N2

    def stage(i, p1):
        sn = i >> 2
        qn = sn & 1
        sb = i & 3

        @pl.when(sb == 0)
        def _():
            soff = ebase + sn * SUPER
            pltpu.sync_copy(row_hbm.at[pl.ds(soff, SUPER)], rowsb.at[qn])
            pltpu.sync_copy(col_hbm.at[pl.ds(soff, SUPER)], colsb.at[qn])

        for g in range(GROUPS):
            sl16 = pl.ds(sb * CHUNK + g * 16, 16)
            c16 = colsb[qn, sl16]
            r16 = rowsb[qn, sl16]
            even = iota16 * 2 + g * 32
            p1v = jnp.full((16,), p1, jnp.int32)
            plsc.store_scatter(gidx, [p1v, even], c16 * 2)
            plsc.store_scatter(gidx, [p1v, even + 1], c16 * 2 + 1)
            ridx[p1, pl.ds(g * 16, 16)] = r16
            cidx[p1, pl.ds(g * 16, 16)] = c16
        pltpu.async_copy(y_hbm.at[gidx.at[p1]], rows.at[pl.ds(128 * p1, 128), :],
                         semr)
        pltpu.async_copy(dsi_hbm.at[ridx.at[p1]], dsir.at[p1], sems)
        pltpu.async_copy(dsi_hbm.at[cidx.at[p1]], dsic.at[p1], sems)

    stage(jnp.int32(0), jnp.int32(0))

    def p3_body(i, carry):
        p = i & 1

        @pl.when(i + 1 < CPT)
        def _():
            stage(i + 1, 1 - p)

        sn = i >> 2
        qn = sn & 1
        sb = i & 3
        pltpu.make_async_copy(dsi_hbm.at[ridx.at[p]], dsir.at[p], sems).wait()
        pltpu.make_async_copy(dsi_hbm.at[cidx.at[p]], dsic.at[p], sems).wait()
        for g in range(GROUPS):
            sl16 = pl.ds(sb * CHUNK + g * 16, 16)
            r16 = rowsb[qn, sl16]
            c16 = colsb[qn, sl16]
            ar, br = _unpack_ab(plsc.load_gather(tab, [r16]))
            ac, bc = _unpack_ab(plsc.load_gather(tab, [c16]))
            m = _tanh16(ar + bc)
            rm = _tanh16(ac + br)
            nd = -(m * rm)
            g16 = pl.ds(g * 16, 16)
            w = dsir[p, g16] * nd * dsic[p, g16]
            loc = r16 - nbase
            inb = (loc >= 0) & (loc < N2)
            wb[g16] = jnp.where(inb, w, 0.0)
            loc = jnp.where(inb, loc, 0)
            even = iota16 * 2 + g * 32
            pv = jnp.full((16,), p, jnp.int32)
            plsc.store_scatter(sidx, [pv, even], loc * 2)
            plsc.store_scatter(sidx, [pv, even + 1], loc * 2 + 1)
        pltpu.make_async_copy(y_hbm.at[gidx.at[p]],
                              rows.at[pl.ds(128 * p, 128), :], semr).wait()

        def scale_body(e, carry2):
            ws = plsc.load_gather(wb, [jnp.full((16,), e, jnp.int32)])
            r0 = 128 * p + 2 * e
            for j in range(128 // 16):
                sl = pl.ds(j * 16, 16)
                rows[r0, sl] = rows[r0, sl] * ws
                rows[r0 + 1, sl] = rows[r0 + 1, sl] * ws
            return carry2

        lax.fori_loop(0, CHUNK, scale_body, 0)
        pltpu.sync_copy(rows.at[pl.ds(128 * p, 128), :], acc.at[sidx.at[p]],
                        add=True)
        return carry

    lax.fori_loop(0, CPT, p3_body, 0)
    plsc.subcore_barrier()

    def combine(gb, lb, nr):
        pltpu.sync_copy(x_hbm.at[pl.ds(2 * gb, 2 * nr), :], rows.at[pl.ds(0, 2 * nr), :])
        pltpu.sync_copy(y_hbm.at[pl.ds(2 * gb, 2 * nr), :], rows.at[pl.ds(32, 2 * nr), :])
        pltpu.sync_copy(acc.at[pl.ds(2 * lb, 2 * nr), :], rows.at[pl.ds(64, 2 * nr), :])
        pltpu.sync_copy(dsc_hbm.at[pl.ds(gb, nr)], dscch.at[pl.ds(0, nr)])

        def rbody(lane, carry2):
            dv = plsc.load_gather(dscch, [jnp.full((16,), lane, jnp.int32)])
            for h in range(2):
                r = 2 * lane + h
                for j in range(128 // 16):
                    sl = pl.ds(j * 16, 16)
                    rows[r, sl] = rows[r, sl] - STEP * (
                        dv * rows[32 + r, sl] + rows[64 + r, sl])
            return carry2

        lax.fori_loop(0, nr, rbody, 0)
        pltpu.sync_copy(rows.at[pl.ds(0, 2 * nr), :], out_hbm.at[pl.ds(2 * gb, 2 * nr), :])

    nfull = jnp.where(s == NSUB - 1, (N2 - (NSUB - 1) * TROWS) // PCH,
                      TROWS // PCH)

    def p4_body(k, carry):
        lb = s * TROWS + k * PCH
        combine(nbase + lb, lb, PCH)
        return carry

    lax.fori_loop(0, nfull, p4_body, 0)

    @pl.when(s == NSUB - 1)
    def _():
        lb = (NSUB - 1) * TROWS + ((N2 - (NSUB - 1) * TROWS) // PCH) * PCH
        combine(nbase + lb, lb, NTAIL)


def _sc_stage(x2, y2, abp, row_p, col_p):
    mesh = plsc.VectorSubcoreMesh(core_axis_name="c", subcore_axis_name="s",
                                  num_cores=NCORE, num_subcores=NSUB)
    f = functools.partial(
        pl.kernel,
        mesh=mesh,
        compiler_params=pltpu.CompilerParams(needs_layout_passes=False),
        out_type=[
            jax.ShapeDtypeStruct((2 * N, 128), jnp.float32),
            jax.ShapeDtypeStruct((N_PAD,), jnp.float32),
            jax.ShapeDtypeStruct((N_PAD,), jnp.float32),
        ],
        scratch_types=[
            pltpu.VMEM((N_PAD,), jnp.int32),
            pltpu.VMEM((2, SUPER), jnp.int32),
            pltpu.VMEM((2, SUPER), jnp.int32),
            pltpu.VMEM((256, 128), jnp.float32),
            pltpu.VMEM((2, 2 * CHUNK), jnp.int32),
            pltpu.VMEM((2, 2 * CHUNK), jnp.int32),
            pltpu.VMEM((2, CHUNK), jnp.int32),
            pltpu.VMEM((2, CHUNK), jnp.int32),
            pltpu.VMEM((2, CHUNK), jnp.float32),
            pltpu.VMEM((2, CHUNK), jnp.float32),
            pltpu.VMEM((CHUNK,), jnp.float32),
            pltpu.VMEM((CHUNK,), jnp.float32),
            pltpu.VMEM((CHUNK,), jnp.int32),
            pltpu.VMEM((N_PAD // NSUB,), jnp.float32),
            pltpu.VMEM((N_PAD // NSUB,), jnp.float32),
            pltpu.VMEM((N_PAD // NSUB,), jnp.float32),
            pltpu.VMEM_SHARED((2 * NSUB * TROWS, 128), jnp.float32),
            pltpu.VMEM_SHARED((N_PAD,), jnp.float32),
            pltpu.SemaphoreType.DMA,
            pltpu.SemaphoreType.DMA,
        ],
    )(_sc_body)
    out2, _, _ = f(x2, y2, abp, row_p, col_p)
    return out2


@jax.jit
def kernel(x, edge_index, W_lin, b_lin, W_sheaf):
    wsp = jnp.zeros((D, 128), jnp.float32)
    wsp = wsp.at[:, 0].set(W_sheaf[:D, 0]).at[:, 1].set(W_sheaf[D:, 0])
    y, abp = _dense(x, W_lin, b_lin.reshape(1, D), wsp)
    a16 = lax.bitcast_convert_type(abp[:, 0].astype(jnp.bfloat16), jnp.uint16)
    b16 = lax.bitcast_convert_type(abp[:, 1].astype(jnp.bfloat16), jnp.uint16)
    packed = (a16.astype(jnp.int32) << 16) | b16.astype(jnp.int32)
    pad = E_PAD - E
    row_p = jnp.concatenate([edge_index[0], jnp.full((pad,), N, jnp.int32)])
    col_p = jnp.concatenate([edge_index[1], jnp.zeros((pad,), jnp.int32)])
    x2 = x.reshape(2 * N, 128)
    y2 = y.reshape(2 * N, 128)
    out2 = _sc_stage(x2, y2, packed, row_p, col_p)
    return out2.reshape(N, D)


# speedup vs baseline: 5.6194x; 1.0012x over previous
import functools

import jax
import jax.numpy as jnp
from jax import lax
from jax.experimental import pallas as pl
from jax.experimental.pallas import tpu as pltpu
import jax.experimental.pallas.tpu_sc as plsc

N = 10000
D = 256
E = 160000
STEP = 0.5

NCORE = 2
NSUB = 16
N2 = N // NCORE
N_PAD = 10240
CHUNK = 64
GROUPS = CHUNK // 16
SUP = 4
SUPER = SUP * CHUNK
SPT = 40
CPT = SPT * SUP
EPT = CPT * CHUNK
E_PAD = EPT * NSUB
TROWS = 320
PCH = 16
NTAIL = 8

MASK_HI = -65536


def _tanh16(z):
    az = jnp.abs(z)
    e = jnp.exp(-2.0 * az)
    p = (1.0 - e) / (1.0 + e)
    return jnp.where(z < 0, -p, p)


def _rsqrt16(u):
    g = plsc.bitcast(jnp.int32(0x5F3759DF) - (plsc.bitcast(u, jnp.int32) >> 1),
                     jnp.float32)
    for _ in range(3):
        g = g * (1.5 - 0.5 * u * g * g)
    return g


def _unpack_ab(v):
    a = plsc.bitcast(v & jnp.int32(MASK_HI), jnp.float32)
    b = plsc.bitcast(v << 16, jnp.float32)
    return a, b


def _dense(x, w_lin, b_lin2, wsp):
    BM = 512
    grid = (pl.cdiv(N, BM),)

    def body(x_ref, wl_ref, bl_ref, ws_ref, y_ref, ab_ref):
        xb = x_ref[...]
        y = lax.dot_general(xb, wl_ref[...], (((1,), (1,)), ((), ())),
                            preferred_element_type=jnp.float32)
        y_ref[...] = y + bl_ref[...]
        ab_ref[...] = jnp.dot(xb, ws_ref[...], preferred_element_type=jnp.float32)

    return pl.pallas_call(
        body,
        grid=grid,
        in_specs=[
            pl.BlockSpec((BM, D), lambda i: (i, 0)),
            pl.BlockSpec((D, D), lambda i: (0, 0)),
            pl.BlockSpec((1, D), lambda i: (0, 0)),
            pl.BlockSpec((D, 128), lambda i: (0, 0)),
        ],
        out_specs=[
            pl.BlockSpec((BM, D), lambda i: (i, 0)),
            pl.BlockSpec((BM, 128), lambda i: (i, 0)),
        ],
        out_shape=[
            jax.ShapeDtypeStruct((N, D), jnp.float32),
            jax.ShapeDtypeStruct((N, 128), jnp.float32),
        ],
    )(x, w_lin, b_lin2, wsp)


def _sc_body(x_hbm, y_hbm, ab_hbm, row_hbm, col_hbm,
             out_hbm, dsi_hbm, dsc_hbm,
             tab, rowsb, colsb, rows, gidx, sidx, rcidx, dsirc,
             wb, m2sb, rowb4,
             dgch, dsich, dscch, acc, dwide, semr, sems):
    c = lax.axis_index("c")
    s = lax.axis_index("s")
    iota16 = lax.iota(jnp.int32, 16)
    zf = jnp.zeros((16,), jnp.float32)
    zi = jnp.zeros((16,), jnp.int32)

    pltpu.sync_copy(ab_hbm, tab.at[pl.ds(0, N)])

    def z_tab(i, carry):
        tab[pl.ds(N + i * 16, 16)] = zi
        return carry

    lax.fori_loop(0, (N_PAD - N) // 16, z_tab, 0)

    def z_rows(r, carry):
        for j in range(128 // 16):
            rows[r, pl.ds(j * 16, 16)] = zf
        return carry

    lax.fori_loop(0, 2 * PCH, z_rows, 0)
    tb = 2 * s * TROWS

    def z_acc(k, carry):
        pltpu.sync_copy(rows.at[pl.ds(0, 2 * PCH), :],
                        acc.at[pl.ds(tb + k * 2 * PCH, 2 * PCH), :])
        return carry

    lax.fori_loop(0, TROWS // PCH, z_acc, 0)
    db = s * (N_PAD // NSUB)

    def z_dsich(i, carry):
        dsich[pl.ds(i * 16, 16)] = zf
        return carry

    lax.fori_loop(0, N_PAD // NSUB // 16, z_dsich, 0)
    pltpu.sync_copy(dsich, dwide.at[pl.ds(db, N_PAD // NSUB)])
    plsc.subcore_barrier()

    ebase = s * EPT

    def p1_super(s2, carry):
        soff = ebase + s2 * SUPER
        pltpu.sync_copy(row_hbm.at[pl.ds(soff, SUPER)], rowsb.at[0])
        pltpu.sync_copy(col_hbm.at[pl.ds(soff, SUPER)], colsb.at[0])

        def p1_sub(k, carry2):
            for g in range(GROUPS):
                sl16 = pl.ds(k * CHUNK + g * 16, 16)
                r16 = rowsb[0, sl16]
                c16 = colsb[0, sl16]
                ar, br = _unpack_ab(plsc.load_gather(tab, [r16]))
                ac, bc = _unpack_ab(plsc.load_gather(tab, [c16]))
                m = _tanh16(ar + bc)
                m2sb[k, pl.ds(g * 16, 16)] = m * m
                rowb4[k, pl.ds(g * 16, 16)] = r16
            pltpu.sync_copy(m2sb.at[k], dwide.at[rowb4.at[k]], add=True)
            return carry2

        lax.fori_loop(0, SUP, p1_sub, 0)
        return carry

    lax.fori_loop(0, SPT, p1_super, 0)
    plsc.subcore_barrier()

    pltpu.sync_copy(dwide.at[pl.ds(db, N_PAD // NSUB)], dgch)

    def p2_body(i, carry):
        v = dgch[pl.ds(i * 16, 16)]
        u = v + 1.0
        dsich[pl.ds(i * 16, 16)] = _rsqrt16(u)
        dscch[pl.ds(i * 16, 16)] = v / u
        return carry

    lax.fori_loop(0, N_PAD // NSUB // 16, p2_body, 0)
    pltpu.sync_copy(dsich, dsi_hbm.at[pl.ds(db, N_PAD // NSUB)])
    pltpu.sync_copy(dscch, dsc_hbm.at[pl.ds(db, N_PAD // NSUB)])
    plsc.subcore_barrier()

    nbase = c * N2

    def stage(i, p1):
        sn = i >> 2
        qn = sn & 1
        sb = i & 3

        @pl.when(sb == 0)
        def _():
            soff = ebase + sn * SUPER
            pltpu.sync_copy(row_hbm.at[pl.ds(soff, SUPER)], rowsb.at[qn])
            pltpu.sync_copy(col_hbm.at[pl.ds(soff, SUPER)], colsb.at[qn])

        for g in range(GROUPS):
            sl16 = pl.ds(sb * CHUNK + g * 16, 16)
            c16 = colsb[qn, sl16]
            r16 = rowsb[qn, sl16]
            even = iota16 * 2 + g * 32
            p1v = jnp.full((16,), p1, jnp.int32)
            plsc.store_scatter(gidx, [p1v, even], c16 * 2)
            plsc.store_scatter(gidx, [p1v, even + 1], c16 * 2 + 1)
            rcidx[p1, pl.ds(g * 16, 16)] = r16
            rcidx[p1, pl.ds(CHUNK + g * 16, 16)] = c16

        pltpu.async_copy(y_hbm.at[gidx.at[p1]], rows.at[pl.ds(128 * p1, 128), :],
                         semr)
        pltpu.async_copy(dsi_hbm.at[rcidx.at[p1]], dsirc.at[p1], sems)

    stage(jnp.int32(0), jnp.int32(0))

    def p3_body(i, carry):
        p = i & 1

        @pl.when(i + 1 < CPT)
        def _():
            stage(i + 1, 1 - p)

        sn = i >> 2
        qn = sn & 1
        sb = i & 3
        pltpu.make_async_copy(dsi_hbm.at[rcidx.at[p]], dsirc.at[p], sems).wait()
        for g in range(GROUPS):
            sl16 = pl.ds(sb * CHUNK + g * 16, 16)
            r16 = rowsb[qn, sl16]
            c16 = colsb[qn, sl16]
            ar, br = _unpack_ab(plsc.load_gather(tab, [r16]))
            ac, bc = _unpack_ab(plsc.load_gather(tab, [c16]))
            m = _tanh16(ar + bc)
            rm = _tanh16(ac + br)
            nd = -(m * rm)
            g16 = pl.ds(g * 16, 16)
            w = dsirc[p, g16] * nd * dsirc[p, pl.ds(CHUNK + g * 16, 16)]
            loc = r16 - nbase
            inb = (loc >= 0) & (loc < N2)
            wb[g16] = jnp.where(inb, w, 0.0)
            loc = jnp.where(inb, loc, 0)
            even = iota16 * 2 + g * 32
            pv = jnp.full((16,), p, jnp.int32)
            plsc.store_scatter(sidx, [pv, even], loc * 2)
            plsc.store_scatter(sidx, [pv, even + 1], loc * 2 + 1)
        pltpu.make_async_copy(y_hbm.at[gidx.at[p]],
                              rows.at[pl.ds(128 * p, 128), :], semr).wait()

        def scale_body(e, carry2):
            ws = plsc.load_gather(wb, [jnp.full((16,), e, jnp.int32)])
            r0 = 128 * p + 2 * e
            for j in range(128 // 16):
                sl = pl.ds(j * 16, 16)
                rows[r0, sl] = rows[r0, sl] * ws
                rows[r0 + 1, sl] = rows[r0 + 1, sl] * ws
            return carry2

        lax.fori_loop(0, CHUNK, scale_body, 0)
        pltpu.sync_copy(rows.at[pl.ds(128 * p, 128), :], acc.at[sidx.at[p]],
                        add=True)
        return carry

    lax.fori_loop(0, CPT, p3_body, 0)
    plsc.subcore_barrier()

    def combine(gb, lb, nr):
        pltpu.sync_copy(x_hbm.at[pl.ds(2 * gb, 2 * nr), :], rows.at[pl.ds(0, 2 * nr), :])
        pltpu.sync_copy(y_hbm.at[pl.ds(2 * gb, 2 * nr), :], rows.at[pl.ds(32, 2 * nr), :])
        pltpu.sync_copy(acc.at[pl.ds(2 * lb, 2 * nr), :], rows.at[pl.ds(64, 2 * nr), :])
        pltpu.sync_copy(dsc_hbm.at[pl.ds(gb, nr)], dscch.at[pl.ds(0, nr)])

        def rbody(lane, carry2):
            dv = plsc.load_gather(dscch, [jnp.full((16,), lane, jnp.int32)])
            for h in range(2):
                r = 2 * lane + h
                for j in range(128 // 16):
                    sl = pl.ds(j * 16, 16)
                    rows[r, sl] = rows[r, sl] - STEP * (
                        dv * rows[32 + r, sl] + rows[64 + r, sl])
            return carry2

        lax.fori_loop(0, nr, rbody, 0)
        pltpu.sync_copy(rows.at[pl.ds(0, 2 * nr), :], out_hbm.at[pl.ds(2 * gb, 2 * nr), :])

    nfull = jnp.where(s == NSUB - 1, (N2 - (NSUB - 1) * TROWS) // PCH,
                      TROWS // PCH)

    def p4_body(k, carry):
        lb = s * TROWS + k * PCH
        combine(nbase + lb, lb, PCH)
        return carry

    lax.fori_loop(0, nfull, p4_body, 0)

    @pl.when(s == NSUB - 1)
    def _():
        lb = (NSUB - 1) * TROWS + ((N2 - (NSUB - 1) * TROWS) // PCH) * PCH
        combine(nbase + lb, lb, NTAIL)


def _sc_stage(x2, y2, abp, row_p, col_p):
    mesh = plsc.VectorSubcoreMesh(core_axis_name="c", subcore_axis_name="s",
                                  num_cores=NCORE, num_subcores=NSUB)
    f = functools.partial(
        pl.kernel,
        mesh=mesh,
        compiler_params=pltpu.CompilerParams(needs_layout_passes=False),
        out_type=[
            jax.ShapeDtypeStruct((2 * N, 128), jnp.float32),
            jax.ShapeDtypeStruct((N_PAD,), jnp.float32),
            jax.ShapeDtypeStruct((N_PAD,), jnp.float32),
        ],
        scratch_types=[
            pltpu.VMEM((N_PAD,), jnp.int32),
            pltpu.VMEM((2, SUPER), jnp.int32),
            pltpu.VMEM((2, SUPER), jnp.int32),
            pltpu.VMEM((256, 128), jnp.float32),
            pltpu.VMEM((2, 2 * CHUNK), jnp.int32),
            pltpu.VMEM((2, 2 * CHUNK), jnp.int32),
            pltpu.VMEM((2, 2 * CHUNK), jnp.int32),
            pltpu.VMEM((2, 2 * CHUNK), jnp.float32),
            pltpu.VMEM((CHUNK,), jnp.float32),
            pltpu.VMEM((SUP, CHUNK), jnp.float32),
            pltpu.VMEM((SUP, CHUNK), jnp.int32),
            pltpu.VMEM((N_PAD // NSUB,), jnp.float32),
            pltpu.VMEM((N_PAD // NSUB,), jnp.float32),
            pltpu.VMEM((N_PAD // NSUB,), jnp.float32),
            pltpu.VMEM_SHARED((2 * NSUB * TROWS, 128), jnp.float32),
            pltpu.VMEM_SHARED((N_PAD,), jnp.float32),
            pltpu.SemaphoreType.DMA,
            pltpu.SemaphoreType.DMA,
        ],
    )(_sc_body)
    out2, _, _ = f(x2, y2, abp, row_p, col_p)
    return out2


@jax.jit
def kernel(x, edge_index, W_lin, b_lin, W_sheaf):
    wsp = jnp.zeros((D, 128), jnp.float32)
    wsp = wsp.at[:, 0].set(W_sheaf[:D, 0]).at[:, 1].set(W_sheaf[D:, 0])
    y, abp = _dense(x, W_lin, b_lin.reshape(1, D), wsp)
    a16 = lax.bitcast_convert_type(abp[:, 0].astype(jnp.bfloat16), jnp.uint16)
    b16 = lax.bitcast_convert_type(abp[:, 1].astype(jnp.bfloat16), jnp.uint16)
    packed = (a16.astype(jnp.int32) << 16) | b16.astype(jnp.int32)
    pad = E_PAD - E
    row_p = jnp.concatenate([edge_index[0], jnp.full((pad,), N, jnp.int32)])
    col_p = jnp.concatenate([edge_index[1], jnp.zeros((pad,), jnp.int32)])
    x2 = x.reshape(2 * N, 128)
    y2 = y.reshape(2 * N, 128)
    out2 = _sc_stage(x2, y2, packed, row_p, col_p)
    return out2.reshape(N, D)


# speedup vs baseline: 5.7278x; 1.0193x over previous
import functools

import jax
import jax.numpy as jnp
from jax import lax
from jax.experimental import pallas as pl
from jax.experimental.pallas import tpu as pltpu
import jax.experimental.pallas.tpu_sc as plsc

N = 10000
D = 256
E = 160000
STEP = 0.5

NCORE = 2
NSUB = 16
N2 = N // NCORE
N_PAD = 10240
CHUNK = 64
GROUPS = CHUNK // 16
SUP = 4
SUPER = SUP * CHUNK
SPT = 40
CPT = SPT * SUP
EPT = CPT * CHUNK
E_PAD = EPT * NSUB
TROWS = 320
PCH = 16
NTAIL = 8

MASK_HI = -65536


def _tanh16(z):
    az = jnp.abs(z)
    e = jnp.exp(-2.0 * az)
    p = (1.0 - e) / (1.0 + e)
    return jnp.where(z < 0, -p, p)


def _rsqrt16(u):
    g = plsc.bitcast(jnp.int32(0x5F3759DF) - (plsc.bitcast(u, jnp.int32) >> 1),
                     jnp.float32)
    for _ in range(3):
        g = g * (1.5 - 0.5 * u * g * g)
    return g


def _unpack_ab(v):
    a = plsc.bitcast(v & jnp.int32(MASK_HI), jnp.float32)
    b = plsc.bitcast(v << 16, jnp.float32)
    return a, b


def _dense(x, w_lin, b_lin2, wsp):
    BM = 512
    grid = (pl.cdiv(N, BM),)

    def body(x_ref, wl_ref, bl_ref, ws_ref, y_ref, ab_ref):
        xb = x_ref[...]
        y = lax.dot_general(xb, wl_ref[...], (((1,), (1,)), ((), ())),
                            preferred_element_type=jnp.float32)
        y_ref[...] = y + bl_ref[...]
        ab_ref[...] = jnp.dot(xb, ws_ref[...], preferred_element_type=jnp.float32)

    return pl.pallas_call(
        body,
        grid=grid,
        in_specs=[
            pl.BlockSpec((BM, D), lambda i: (i, 0)),
            pl.BlockSpec((D, D), lambda i: (0, 0)),
            pl.BlockSpec((1, D), lambda i: (0, 0)),
            pl.BlockSpec((D, 128), lambda i: (0, 0)),
        ],
        out_specs=[
            pl.BlockSpec((BM, D), lambda i: (i, 0)),
            pl.BlockSpec((BM, 128), lambda i: (i, 0)),
        ],
        out_shape=[
            jax.ShapeDtypeStruct((N, D), jnp.float32),
            jax.ShapeDtypeStruct((N, 128), jnp.float32),
        ],
    )(x, w_lin, b_lin2, wsp)


def _sc_body(x_hbm, y_hbm, ab_hbm, row_hbm, col_hbm,
             out_hbm, dsi_hbm, dsc_hbm,
             tab, rowsb, colsb, rows, gidx, sidx, rcidx, dsirc,
             wb, m2sb, rowb4,
             dgch, dsich, dscch, acc, dwide, semr, sems):
    c = lax.axis_index("c")
    s = lax.axis_index("s")
    iota16 = lax.iota(jnp.int32, 16)
    zf = jnp.zeros((16,), jnp.float32)
    zi = jnp.zeros((16,), jnp.int32)

    pltpu.sync_copy(ab_hbm, tab.at[pl.ds(0, N)])

    def z_tab(i, carry):
        tab[pl.ds(N + i * 16, 16)] = zi
        return carry

    lax.fori_loop(0, (N_PAD - N) // 16, z_tab, 0)

    def z_rows(r, carry):
        for j in range(128 // 16):
            rows[r, pl.ds(j * 16, 16)] = zf
        return carry

    lax.fori_loop(0, 2 * PCH, z_rows, 0)
    tb = 2 * s * TROWS

    def z_acc(k, carry):
        pltpu.sync_copy(rows.at[pl.ds(0, 2 * PCH), :],
                        acc.at[pl.ds(tb + k * 2 * PCH, 2 * PCH), :])
        return carry

    lax.fori_loop(0, TROWS // PCH, z_acc, 0)
    db = s * (N_PAD // NSUB)

    def z_dsich(i, carry):
        dsich[pl.ds(i * 16, 16)] = zf
        return carry

    lax.fori_loop(0, N_PAD // NSUB // 16, z_dsich, 0)
    pltpu.sync_copy(dsich, dwide.at[pl.ds(db, N_PAD // NSUB)])
    plsc.subcore_barrier()

    ebase = s * EPT

    def p1_super(s2, carry):
        soff = ebase + s2 * SUPER
        pltpu.sync_copy(row_hbm.at[pl.ds(soff, SUPER)], rowsb.at[0])
        pltpu.sync_copy(col_hbm.at[pl.ds(soff, SUPER)], colsb.at[0])

        def p1_sub(h, carry2):
            for g in range(2 * GROUPS):
                sl16 = pl.ds(h * 2 * CHUNK + g * 16, 16)
                r16 = rowsb[0, sl16]
                c16 = colsb[0, sl16]
                ar, br = _unpack_ab(plsc.load_gather(tab, [r16]))
                ac, bc = _unpack_ab(plsc.load_gather(tab, [c16]))
                m = _tanh16(ar + bc)
                m2sb[h, pl.ds(g * 16, 16)] = m * m
                rowb4[h, pl.ds(g * 16, 16)] = r16
            pltpu.sync_copy(m2sb.at[h], dwide.at[rowb4.at[h]], add=True)
            return carry2

        lax.fori_loop(0, SUP // 2, p1_sub, 0)
        return carry

    lax.fori_loop(0, SPT, p1_super, 0)
    plsc.subcore_barrier()

    pltpu.sync_copy(dwide.at[pl.ds(db, N_PAD // NSUB)], dgch)

    def p2_body(i, carry):
        v = dgch[pl.ds(i * 16, 16)]
        u = v + 1.0
        dsich[pl.ds(i * 16, 16)] = _rsqrt16(u)
        dscch[pl.ds(i * 16, 16)] = v / u
        return carry

    lax.fori_loop(0, N_PAD // NSUB // 16, p2_body, 0)
    pltpu.sync_copy(dsich, dsi_hbm.at[pl.ds(db, N_PAD // NSUB)])
    pltpu.sync_copy(dscch, dsc_hbm.at[pl.ds(db, N_PAD // NSUB)])
    plsc.subcore_barrier()

    nbase = c * N2

    def stage(i, p1):
        sn = i >> 2
        qn = sn & 1
        sb = i & 3

        @pl.when(sb == 0)
        def _():
            soff = ebase + sn * SUPER
            pltpu.sync_copy(row_hbm.at[pl.ds(soff, SUPER)], rowsb.at[qn])
            pltpu.sync_copy(col_hbm.at[pl.ds(soff, SUPER)], colsb.at[qn])

        for g in range(GROUPS):
            sl16 = pl.ds(sb * CHUNK + g * 16, 16)
            c16 = colsb[qn, sl16]
            r16 = rowsb[qn, sl16]
            even = iota16 * 2 + g * 32
            p1v = jnp.full((16,), p1, jnp.int32)
            plsc.store_scatter(gidx, [p1v, even], c16 * 2)
            plsc.store_scatter(gidx, [p1v, even + 1], c16 * 2 + 1)
            rcidx[p1, pl.ds(g * 16, 16)] = r16
            rcidx[p1, pl.ds(CHUNK + g * 16, 16)] = c16

        pltpu.async_copy(y_hbm.at[gidx.at[p1]], rows.at[pl.ds(128 * p1, 128), :],
                         semr)
        pltpu.async_copy(dsi_hbm.at[rcidx.at[p1]], dsirc.at[p1], sems)

    stage(jnp.int32(0), jnp.int32(0))

    def p3_body(i, carry):
        p = i & 1

        @pl.when(i + 1 < CPT)
        def _():
            stage(i + 1, 1 - p)

        sn = i >> 2
        qn = sn & 1
        sb = i & 3
        pltpu.make_async_copy(dsi_hbm.at[rcidx.at[p]], dsirc.at[p], sems).wait()
        for g in range(GROUPS):
            sl16 = pl.ds(sb * CHUNK + g * 16, 16)
            r16 = rowsb[qn, sl16]
            c16 = colsb[qn, sl16]
            ar, br = _unpack_ab(plsc.load_gather(tab, [r16]))
            ac, bc = _unpack_ab(plsc.load_gather(tab, [c16]))
            m = _tanh16(ar + bc)
            rm = _tanh16(ac + br)
            nd = -(m * rm)
            g16 = pl.ds(g * 16, 16)
            w = dsirc[p, g16] * nd * dsirc[p, pl.ds(CHUNK + g * 16, 16)]
            loc = r16 - nbase
            inb = (loc >= 0) & (loc < N2)
            wb[g16] = jnp.where(inb, w, 0.0)
            loc = jnp.where(inb, loc, 0)
            even = iota16 * 2 + g * 32
            pv = jnp.full((16,), p, jnp.int32)
            plsc.store_scatter(sidx, [pv, even], loc * 2)
            plsc.store_scatter(sidx, [pv, even + 1], loc * 2 + 1)
        pltpu.make_async_copy(y_hbm.at[gidx.at[p]],
                              rows.at[pl.ds(128 * p, 128), :], semr).wait()

        def scale_body(e2, carry2):
            e = 2 * e2
            ws0 = plsc.load_gather(wb, [jnp.full((16,), e, jnp.int32)])
            ws1 = plsc.load_gather(wb, [jnp.full((16,), e + 1, jnp.int32)])
            r0 = 128 * p + 2 * e
            for j in range(128 // 16):
                sl = pl.ds(j * 16, 16)
                rows[r0, sl] = rows[r0, sl] * ws0
                rows[r0 + 1, sl] = rows[r0 + 1, sl] * ws0
                rows[r0 + 2, sl] = rows[r0 + 2, sl] * ws1
                rows[r0 + 3, sl] = rows[r0 + 3, sl] * ws1
            return carry2

        lax.fori_loop(0, CHUNK // 2, scale_body, 0)
        pltpu.sync_copy(rows.at[pl.ds(128 * p, 128), :], acc.at[sidx.at[p]],
                        add=True)
        return carry

    lax.fori_loop(0, CPT, p3_body, 0)
    plsc.subcore_barrier()

    def combine(gb, lb, nr):
        pltpu.sync_copy(x_hbm.at[pl.ds(2 * gb, 2 * nr), :], rows.at[pl.ds(0, 2 * nr), :])
        pltpu.sync_copy(y_hbm.at[pl.ds(2 * gb, 2 * nr), :], rows.at[pl.ds(32, 2 * nr), :])
        pltpu.sync_copy(acc.at[pl.ds(2 * lb, 2 * nr), :], rows.at[pl.ds(64, 2 * nr), :])
        pltpu.sync_copy(dsc_hbm.at[pl.ds(gb, nr)], dscch.at[pl.ds(0, nr)])

        def rbody(lane, carry2):
            dv = plsc.load_gather(dscch, [jnp.full((16,), lane, jnp.int32)])
            for h in range(2):
                r = 2 * lane + h
                for j in range(128 // 16):
                    sl = pl.ds(j * 16, 16)
                    rows[r, sl] = rows[r, sl] - STEP * (
                        dv * rows[32 + r, sl] + rows[64 + r, sl])
            return carry2

        lax.fori_loop(0, nr, rbody, 0)
        pltpu.sync_copy(rows.at[pl.ds(0, 2 * nr), :], out_hbm.at[pl.ds(2 * gb, 2 * nr), :])

    nfull = jnp.where(s == NSUB - 1, (N2 - (NSUB - 1) * TROWS) // PCH,
                      TROWS // PCH)

    def p4_body(k, carry):
        lb = s * TROWS + k * PCH
        combine(nbase + lb, lb, PCH)
        return carry

    lax.fori_loop(0, nfull, p4_body, 0)

    @pl.when(s == NSUB - 1)
    def _():
        lb = (NSUB - 1) * TROWS + ((N2 - (NSUB - 1) * TROWS) // PCH) * PCH
        combine(nbase + lb, lb, NTAIL)


def _sc_stage(x2, y2, abp, row_p, col_p):
    mesh = plsc.VectorSubcoreMesh(core_axis_name="c", subcore_axis_name="s",
                                  num_cores=NCORE, num_subcores=NSUB)
    f = functools.partial(
        pl.kernel,
        mesh=mesh,
        compiler_params=pltpu.CompilerParams(needs_layout_passes=False),
        out_type=[
            jax.ShapeDtypeStruct((2 * N, 128), jnp.float32),
            jax.ShapeDtypeStruct((N_PAD,), jnp.float32),
            jax.ShapeDtypeStruct((N_PAD,), jnp.float32),
        ],
        scratch_types=[
            pltpu.VMEM((N_PAD,), jnp.int32),
            pltpu.VMEM((2, SUPER), jnp.int32),
            pltpu.VMEM((2, SUPER), jnp.int32),
            pltpu.VMEM((256, 128), jnp.float32),
            pltpu.VMEM((2, 2 * CHUNK), jnp.int32),
            pltpu.VMEM((2, 2 * CHUNK), jnp.int32),
            pltpu.VMEM((2, 2 * CHUNK), jnp.int32),
            pltpu.VMEM((2, 2 * CHUNK), jnp.float32),
            pltpu.VMEM((CHUNK,), jnp.float32),
            pltpu.VMEM((2, 2 * CHUNK), jnp.float32),
            pltpu.VMEM((2, 2 * CHUNK), jnp.int32),
            pltpu.VMEM((N_PAD // NSUB,), jnp.float32),
            pltpu.VMEM((N_PAD // NSUB,), jnp.float32),
            pltpu.VMEM((N_PAD // NSUB,), jnp.float32),
            pltpu.VMEM_SHARED((2 * NSUB * TROWS, 128), jnp.float32),
            pltpu.VMEM_SHARED((N_PAD,), jnp.float32),
            pltpu.SemaphoreType.DMA,
            pltpu.SemaphoreType.DMA,
        ],
    )(_sc_body)
    out2, _, _ = f(x2, y2, abp, row_p, col_p)
    return out2


@jax.jit
def kernel(x, edge_index, W_lin, b_lin, W_sheaf):
    wsp = jnp.zeros((D, 128), jnp.float32)
    wsp = wsp.at[:, 0].set(W_sheaf[:D, 0]).at[:, 1].set(W_sheaf[D:, 0])
    y, abp = _dense(x, W_lin, b_lin.reshape(1, D), wsp)
    a16 = lax.bitcast_convert_type(abp[:, 0].astype(jnp.bfloat16), jnp.uint16)
    b16 = lax.bitcast_convert_type(abp[:, 1].astype(jnp.bfloat16), jnp.uint16)
    packed = (a16.astype(jnp.int32) << 16) | b16.astype(jnp.int32)
    pad = E_PAD - E
    row_p = jnp.concatenate([edge_index[0], jnp.full((pad,), N, jnp.int32)])
    col_p = jnp.concatenate([edge_index[1], jnp.zeros((pad,), jnp.int32)])
    x2 = x.reshape(2 * N, 128)
    y2 = y.reshape(2 * N, 128)
    out2 = _sc_stage(x2, y2, packed, row_p, col_p)
    return out2.reshape(N, D)


# speedup vs baseline: 5.7782x; 1.0088x over previous
import functools

import jax
import jax.numpy as jnp
from jax import lax
from jax.experimental import pallas as pl
from jax.experimental.pallas import tpu as pltpu
import jax.experimental.pallas.tpu_sc as plsc

N = 10000
D = 256
E = 160000
STEP = 0.5

NCORE = 2
NSUB = 16
N2 = N // NCORE
N_PAD = 10240
CHUNK = 64
GROUPS = CHUNK // 16
SUP = 4
SUPER = SUP * CHUNK
SPT = 40
CPT = SPT * SUP
EPT = CPT * CHUNK
E_PAD = EPT * NSUB
TROWS = 320
PCH = 16
NTAIL = 8

MASK_HI = -65536


def _tanh16(z):
    az = jnp.abs(z)
    e = jnp.exp(-2.0 * az)
    p = (1.0 - e) / (1.0 + e)
    return jnp.where(z < 0, -p, p)


def _rsqrt16(u):
    g = plsc.bitcast(jnp.int32(0x5F3759DF) - (plsc.bitcast(u, jnp.int32) >> 1),
                     jnp.float32)
    for _ in range(3):
        g = g * (1.5 - 0.5 * u * g * g)
    return g


def _unpack_ab(v):
    a = plsc.bitcast(v & jnp.int32(MASK_HI), jnp.float32)
    b = plsc.bitcast(v << 16, jnp.float32)
    return a, b


def _dense(x, w_lin, b_lin2, wsp):
    BM = 512
    grid = (pl.cdiv(N, BM),)

    def body(x_ref, wl_ref, bl_ref, ws_ref, y_ref, ab_ref):
        xb = x_ref[...]
        y = lax.dot_general(xb, wl_ref[...], (((1,), (1,)), ((), ())),
                            preferred_element_type=jnp.float32)
        y_ref[...] = y + bl_ref[...]
        ab_ref[...] = jnp.dot(xb, ws_ref[...], preferred_element_type=jnp.float32)

    return pl.pallas_call(
        body,
        grid=grid,
        in_specs=[
            pl.BlockSpec((BM, D), lambda i: (i, 0)),
            pl.BlockSpec((D, D), lambda i: (0, 0)),
            pl.BlockSpec((1, D), lambda i: (0, 0)),
            pl.BlockSpec((D, 128), lambda i: (0, 0)),
        ],
        out_specs=[
            pl.BlockSpec((BM, D), lambda i: (i, 0)),
            pl.BlockSpec((BM, 128), lambda i: (i, 0)),
        ],
        out_shape=[
            jax.ShapeDtypeStruct((N, D), jnp.float32),
            jax.ShapeDtypeStruct((N, 128), jnp.float32),
        ],
    )(x, w_lin, b_lin2, wsp)


def _sc_body(x_hbm, y_hbm, ab_hbm, row_hbm, col_hbm,
             out_hbm, dsi_hbm, dsc_hbm,
             tab, rowsb, colsb, rows, gidx, sidx, rcidx, dsirc,
             wb, m2sb, rowb4,
             dgch, dsich, dscch, acc, dwide, semr, sems, semw):
    c = lax.axis_index("c")
    s = lax.axis_index("s")
    iota16 = lax.iota(jnp.int32, 16)
    zf = jnp.zeros((16,), jnp.float32)
    zi = jnp.zeros((16,), jnp.int32)

    pltpu.sync_copy(ab_hbm, tab.at[pl.ds(0, N)])

    def z_tab(i, carry):
        tab[pl.ds(N + i * 16, 16)] = zi
        return carry

    lax.fori_loop(0, (N_PAD - N) // 16, z_tab, 0)

    def z_rows(r, carry):
        for j in range(128 // 16):
            rows[r, pl.ds(j * 16, 16)] = zf
        return carry

    lax.fori_loop(0, 2 * PCH, z_rows, 0)
    tb = 2 * s * TROWS

    def z_acc(k, carry):
        pltpu.sync_copy(rows.at[pl.ds(0, 2 * PCH), :],
                        acc.at[pl.ds(tb + k * 2 * PCH, 2 * PCH), :])
        return carry

    lax.fori_loop(0, TROWS // PCH, z_acc, 0)
    db = s * (N_PAD // NSUB)

    def z_dsich(i, carry):
        dsich[pl.ds(i * 16, 16)] = zf
        return carry

    lax.fori_loop(0, N_PAD // NSUB // 16, z_dsich, 0)
    pltpu.sync_copy(dsich, dwide.at[pl.ds(db, N_PAD // NSUB)])
    plsc.subcore_barrier()

    ebase = s * EPT

    def p1_super(s2, carry):
        soff = ebase + s2 * SUPER
        pltpu.sync_copy(row_hbm.at[pl.ds(soff, SUPER)], rowsb.at[0])
        pltpu.sync_copy(col_hbm.at[pl.ds(soff, SUPER)], colsb.at[0])

        def p1_sub(h, carry2):
            for g in range(2 * GROUPS):
                sl16 = pl.ds(h * 2 * CHUNK + g * 16, 16)
                r16 = rowsb[0, sl16]
                c16 = colsb[0, sl16]
                ar, br = _unpack_ab(plsc.load_gather(tab, [r16]))
                ac, bc = _unpack_ab(plsc.load_gather(tab, [c16]))
                m = _tanh16(ar + bc)
                m2sb[h, pl.ds(g * 16, 16)] = m * m
                rowb4[h, pl.ds(g * 16, 16)] = r16
            pltpu.sync_copy(m2sb.at[h], dwide.at[rowb4.at[h]], add=True)
            return carry2

        lax.fori_loop(0, SUP // 2, p1_sub, 0)
        return carry

    lax.fori_loop(0, SPT, p1_super, 0)
    plsc.subcore_barrier()

    pltpu.sync_copy(dwide.at[pl.ds(db, N_PAD // NSUB)], dgch)

    def p2_body(i, carry):
        v = dgch[pl.ds(i * 16, 16)]
        u = v + 1.0
        dsich[pl.ds(i * 16, 16)] = _rsqrt16(u)
        dscch[pl.ds(i * 16, 16)] = v / u
        return carry

    lax.fori_loop(0, N_PAD // NSUB // 16, p2_body, 0)
    pltpu.sync_copy(dsich, dsi_hbm.at[pl.ds(db, N_PAD // NSUB)])
    pltpu.sync_copy(dscch, dsc_hbm.at[pl.ds(db, N_PAD // NSUB)])
    plsc.subcore_barrier()

    nbase = c * N2

    def stage(i, p1):
        sn = i >> 2
        qn = sn & 1
        sb = i & 3

        @pl.when(sb == 0)
        def _():
            soff = ebase + sn * SUPER
            pltpu.sync_copy(row_hbm.at[pl.ds(soff, SUPER)], rowsb.at[qn])
            pltpu.sync_copy(col_hbm.at[pl.ds(soff, SUPER)], colsb.at[qn])

        for g in range(GROUPS):
            sl16 = pl.ds(sb * CHUNK + g * 16, 16)
            c16 = colsb[qn, sl16]
            r16 = rowsb[qn, sl16]
            even = iota16 * 2 + g * 32
            p1v = jnp.full((16,), p1, jnp.int32)
            plsc.store_scatter(gidx, [p1v, even], c16 * 2)
            plsc.store_scatter(gidx, [p1v, even + 1], c16 * 2 + 1)
            rcidx[p1, pl.ds(g * 16, 16)] = r16
            rcidx[p1, pl.ds(CHUNK + g * 16, 16)] = c16

        pltpu.async_copy(y_hbm.at[gidx.at[p1]], rows.at[pl.ds(128 * p1, 128), :],
                         semr)
        pltpu.async_copy(dsi_hbm.at[rcidx.at[p1]], dsirc.at[p1], sems)

    stage(jnp.int32(0), jnp.int32(0))

    def weights_chunk(i, p):
        sn = i >> 2
        qn = sn & 1
        sb = i & 3
        pltpu.make_async_copy(dsi_hbm.at[rcidx.at[p]], dsirc.at[p], sems).wait()
        for g in range(GROUPS):
            sl16 = pl.ds(sb * CHUNK + g * 16, 16)
            r16 = rowsb[qn, sl16]
            c16 = colsb[qn, sl16]
            ar, br = _unpack_ab(plsc.load_gather(tab, [r16]))
            ac, bc = _unpack_ab(plsc.load_gather(tab, [c16]))
            m = _tanh16(ar + bc)
            rm = _tanh16(ac + br)
            nd = -(m * rm)
            g16 = pl.ds(g * 16, 16)
            w = dsirc[p, g16] * nd * dsirc[p, pl.ds(CHUNK + g * 16, 16)]
            loc = r16 - nbase
            inb = (loc >= 0) & (loc < N2)
            wb[g16] = jnp.where(inb, w, 0.0)
            loc = jnp.where(inb, loc, 0)
            even = iota16 * 2 + g * 32
            pv = jnp.full((16,), p, jnp.int32)
            plsc.store_scatter(sidx, [pv, even], loc * 2)
            plsc.store_scatter(sidx, [pv, even + 1], loc * 2 + 1)

    def scale_slot(p):
        def scale_body(e2, carry2):
            e = 2 * e2
            ws0 = plsc.load_gather(wb, [jnp.full((16,), e, jnp.int32)])
            ws1 = plsc.load_gather(wb, [jnp.full((16,), e + 1, jnp.int32)])
            r0 = 128 * p + 2 * e
            for j in range(128 // 16):
                sl = pl.ds(j * 16, 16)
                rows[r0, sl] = rows[r0, sl] * ws0
                rows[r0 + 1, sl] = rows[r0 + 1, sl] * ws0
                rows[r0 + 2, sl] = rows[r0 + 2, sl] * ws1
                rows[r0 + 3, sl] = rows[r0 + 3, sl] * ws1
            return carry2

        lax.fori_loop(0, CHUNK // 2, scale_body, 0)

    def p3_pair(i2, carry):
        i = 2 * i2
        stage(i + 1, jnp.int32(1))
        weights_chunk(i, 0)
        pltpu.make_async_copy(y_hbm.at[gidx.at[0]],
                              rows.at[pl.ds(0, 128), :], semr).wait()
        scale_slot(0)
        d0 = pltpu.async_copy(rows.at[pl.ds(0, 128), :], acc.at[sidx.at[0]],
                              semw, add=True)
        weights_chunk(i + 1, 1)
        d0.wait()

        @pl.when(i + 2 < CPT)
        def _():
            stage(i + 2, jnp.int32(0))

        pltpu.make_async_copy(y_hbm.at[gidx.at[1]],
                              rows.at[pl.ds(128, 128), :], semr).wait()
        scale_slot(1)
        pltpu.sync_copy(rows.at[pl.ds(128, 128), :], acc.at[sidx.at[1]],
                        add=True)
        return carry

    lax.fori_loop(0, CPT // 2, p3_pair, 0)
    plsc.subcore_barrier()

    def combine(gb, lb, nr):
        pltpu.sync_copy(x_hbm.at[pl.ds(2 * gb, 2 * nr), :], rows.at[pl.ds(0, 2 * nr), :])
        pltpu.sync_copy(y_hbm.at[pl.ds(2 * gb, 2 * nr), :], rows.at[pl.ds(32, 2 * nr), :])
        pltpu.sync_copy(acc.at[pl.ds(2 * lb, 2 * nr), :], rows.at[pl.ds(64, 2 * nr), :])
        pltpu.sync_copy(dsc_hbm.at[pl.ds(gb, nr)], dscch.at[pl.ds(0, nr)])

        def rbody(lane, carry2):
            dv = plsc.load_gather(dscch, [jnp.full((16,), lane, jnp.int32)])
            for h in range(2):
                r = 2 * lane + h
                for j in range(128 // 16):
                    sl = pl.ds(j * 16, 16)
                    rows[r, sl] = rows[r, sl] - STEP * (
                        dv * rows[32 + r, sl] + rows[64 + r, sl])
            return carry2

        lax.fori_loop(0, nr, rbody, 0)
        pltpu.sync_copy(rows.at[pl.ds(0, 2 * nr), :], out_hbm.at[pl.ds(2 * gb, 2 * nr), :])

    nfull = jnp.where(s == NSUB - 1, (N2 - (NSUB - 1) * TROWS) // PCH,
                      TROWS // PCH)

    def p4_body(k, carry):
        lb = s * TROWS + k * PCH
        combine(nbase + lb, lb, PCH)
        return carry

    lax.fori_loop(0, nfull, p4_body, 0)

    @pl.when(s == NSUB - 1)
    def _():
        lb = (NSUB - 1) * TROWS + ((N2 - (NSUB - 1) * TROWS) // PCH) * PCH
        combine(nbase + lb, lb, NTAIL)


def _sc_stage(x2, y2, abp, row_p, col_p):
    mesh = plsc.VectorSubcoreMesh(core_axis_name="c", subcore_axis_name="s",
                                  num_cores=NCORE, num_subcores=NSUB)
    f = functools.partial(
        pl.kernel,
        mesh=mesh,
        compiler_params=pltpu.CompilerParams(needs_layout_passes=False),
        out_type=[
            jax.ShapeDtypeStruct((2 * N, 128), jnp.float32),
            jax.ShapeDtypeStruct((N_PAD,), jnp.float32),
            jax.ShapeDtypeStruct((N_PAD,), jnp.float32),
        ],
        scratch_types=[
            pltpu.VMEM((N_PAD,), jnp.int32),
            pltpu.VMEM((2, SUPER), jnp.int32),
            pltpu.VMEM((2, SUPER), jnp.int32),
            pltpu.VMEM((256, 128), jnp.float32),
            pltpu.VMEM((2, 2 * CHUNK), jnp.int32),
            pltpu.VMEM((2, 2 * CHUNK), jnp.int32),
            pltpu.VMEM((2, 2 * CHUNK), jnp.int32),
            pltpu.VMEM((2, 2 * CHUNK), jnp.float32),
            pltpu.VMEM((CHUNK,), jnp.float32),
            pltpu.VMEM((2, 2 * CHUNK), jnp.float32),
            pltpu.VMEM((2, 2 * CHUNK), jnp.int32),
            pltpu.VMEM((N_PAD // NSUB,), jnp.float32),
            pltpu.VMEM((N_PAD // NSUB,), jnp.float32),
            pltpu.VMEM((N_PAD // NSUB,), jnp.float32),
            pltpu.VMEM_SHARED((2 * NSUB * TROWS, 128), jnp.float32),
            pltpu.VMEM_SHARED((N_PAD,), jnp.float32),
            pltpu.SemaphoreType.DMA,
            pltpu.SemaphoreType.DMA,
            pltpu.SemaphoreType.DMA,
        ],
    )(_sc_body)
    out2, _, _ = f(x2, y2, abp, row_p, col_p)
    return out2


@jax.jit
def kernel(x, edge_index, W_lin, b_lin, W_sheaf):
    wsp = jnp.zeros((D, 128), jnp.float32)
    wsp = wsp.at[:, 0].set(W_sheaf[:D, 0]).at[:, 1].set(W_sheaf[D:, 0])
    y, abp = _dense(x, W_lin, b_lin.reshape(1, D), wsp)
    a16 = lax.bitcast_convert_type(abp[:, 0].astype(jnp.bfloat16), jnp.uint16)
    b16 = lax.bitcast_convert_type(abp[:, 1].astype(jnp.bfloat16), jnp.uint16)
    packed = (a16.astype(jnp.int32) << 16) | b16.astype(jnp.int32)
    pad = E_PAD - E
    row_p = jnp.concatenate([edge_index[0], jnp.full((pad,), N, jnp.int32)])
    col_p = jnp.concatenate([edge_index[1], jnp.zeros((pad,), jnp.int32)])
    x2 = x.reshape(2 * N, 128)
    y2 = y.reshape(2 * N, 128)
    out2 = _sc_stage(x2, y2, packed, row_p, col_p)
    return out2.reshape(N, D)


# speedup vs baseline: 6.2345x; 1.0790x over previous
import functools

import jax
import jax.numpy as jnp
from jax import lax
from jax.experimental import pallas as pl
from jax.experimental.pallas import tpu as pltpu
import jax.experimental.pallas.tpu_sc as plsc

N = 10000
D = 256
E = 160000
STEP = 0.5

NCORE = 2
NSUB = 16
N2 = N // NCORE
N_PAD = 10240
CHUNK = 64
GROUPS = CHUNK // 16
SUP = 4
SUPER = SUP * CHUNK
SPT = 40
CPT = SPT * SUP
EPT = CPT * CHUNK
E_PAD = EPT * NSUB
TROWS = 320
PCH = 32
NTAIL = 8

MASK_HI = -65536


def _tanh16(z):
    az = jnp.abs(z)
    e = jnp.exp(-2.0 * az)
    p = (1.0 - e) / (1.0 + e)
    return jnp.where(z < 0, -p, p)


def _rsqrt16(u):
    g = plsc.bitcast(jnp.int32(0x5F3759DF) - (plsc.bitcast(u, jnp.int32) >> 1),
                     jnp.float32)
    for _ in range(3):
        g = g * (1.5 - 0.5 * u * g * g)
    return g


def _unpack_ab(v):
    a = plsc.bitcast(v & jnp.int32(MASK_HI), jnp.float32)
    b = plsc.bitcast(v << 16, jnp.float32)
    return a, b


def _dense(x, w_lin, b_lin2, wsp):
    BM = 512
    grid = (pl.cdiv(N, BM),)

    def body(x_ref, wl_ref, bl_ref, ws_ref, y_ref, ab_ref):
        xb = x_ref[...]
        y = lax.dot_general(xb, wl_ref[...], (((1,), (1,)), ((), ())),
                            preferred_element_type=jnp.float32)
        y_ref[...] = y + bl_ref[...]
        ab_ref[...] = jnp.dot(xb, ws_ref[...], preferred_element_type=jnp.float32)

    return pl.pallas_call(
        body,
        grid=grid,
        in_specs=[
            pl.BlockSpec((BM, D), lambda i: (i, 0)),
            pl.BlockSpec((D, D), lambda i: (0, 0)),
            pl.BlockSpec((1, D), lambda i: (0, 0)),
            pl.BlockSpec((D, 128), lambda i: (0, 0)),
        ],
        out_specs=[
            pl.BlockSpec((BM, D), lambda i: (i, 0)),
            pl.BlockSpec((BM, 128), lambda i: (i, 0)),
        ],
        out_shape=[
            jax.ShapeDtypeStruct((N, D), jnp.float32),
            jax.ShapeDtypeStruct((N, 128), jnp.float32),
        ],
    )(x, w_lin, b_lin2, wsp)


def _sc_body(x_hbm, y_hbm, ab_hbm, row_hbm, col_hbm,
             out_hbm, dsi_hbm, dsc_hbm,
             tab, rowsb, colsb, rows, gidx, sidx, rcidx, dsirc,
             wb, m2sb, rowb4,
             dgch, dsich, dscch, acc, dwide, semr, sems, semw):
    c = lax.axis_index("c")
    s = lax.axis_index("s")
    iota16 = lax.iota(jnp.int32, 16)
    zf = jnp.zeros((16,), jnp.float32)
    zi = jnp.zeros((16,), jnp.int32)

    pltpu.sync_copy(ab_hbm, tab.at[pl.ds(0, N)])

    def z_tab(i, carry):
        tab[pl.ds(N + i * 16, 16)] = zi
        return carry

    lax.fori_loop(0, (N_PAD - N) // 16, z_tab, 0)

    def z_rows(r, carry):
        for j in range(128 // 16):
            rows[r, pl.ds(j * 16, 16)] = zf
        return carry

    lax.fori_loop(0, 2 * PCH, z_rows, 0)
    tb = 2 * s * TROWS

    def z_acc(k, carry):
        pltpu.sync_copy(rows.at[pl.ds(0, 2 * PCH), :],
                        acc.at[pl.ds(tb + k * 2 * PCH, 2 * PCH), :])
        return carry

    lax.fori_loop(0, TROWS // PCH, z_acc, 0)
    db = s * (N_PAD // NSUB)

    def z_dsich(i, carry):
        dsich[pl.ds(i * 16, 16)] = zf
        return carry

    lax.fori_loop(0, N_PAD // NSUB // 16, z_dsich, 0)
    pltpu.sync_copy(dsich, dwide.at[pl.ds(db, N_PAD // NSUB)])
    plsc.subcore_barrier()

    ebase = s * EPT

    def p1_fire_meta(s2, q):
        soff = ebase + s2 * SUPER
        pltpu.async_copy(row_hbm.at[pl.ds(soff, SUPER)], rowsb.at[q], sems)
        pltpu.async_copy(col_hbm.at[pl.ds(soff, SUPER)], colsb.at[q], sems)

    p1_fire_meta(jnp.int32(0), jnp.int32(0))

    def p1_super(s2, carry):
        q = s2 & 1
        pltpu.make_async_copy(row_hbm.at[pl.ds(0, SUPER)], rowsb.at[q],
                              sems).wait()
        pltpu.make_async_copy(col_hbm.at[pl.ds(0, SUPER)], colsb.at[q],
                              sems).wait()

        @pl.when(s2 + 1 < SPT)
        def _():
            p1_fire_meta(s2 + 1, 1 - q)

        def p1_sub(h, carry2):
            for g in range(2 * GROUPS):
                sl16 = pl.ds(h * 2 * CHUNK + g * 16, 16)
                r16 = rowsb[q, sl16]
                c16 = colsb[q, sl16]
                ar, br = _unpack_ab(plsc.load_gather(tab, [r16]))
                ac, bc = _unpack_ab(plsc.load_gather(tab, [c16]))
                m = _tanh16(ar + bc)
                m2sb[h, pl.ds(g * 16, 16)] = m * m
                rowb4[h, pl.ds(g * 16, 16)] = r16
            pltpu.sync_copy(m2sb.at[h], dwide.at[rowb4.at[h]], add=True)
            return carry2

        lax.fori_loop(0, SUP // 2, p1_sub, 0)
        return carry

    lax.fori_loop(0, SPT, p1_super, 0)
    plsc.subcore_barrier()

    pltpu.sync_copy(dwide.at[pl.ds(db, N_PAD // NSUB)], dgch)

    def p2_body(i, carry):
        v = dgch[pl.ds(i * 16, 16)]
        u = v + 1.0
        dsich[pl.ds(i * 16, 16)] = _rsqrt16(u)
        dscch[pl.ds(i * 16, 16)] = v / u
        return carry

    lax.fori_loop(0, N_PAD // NSUB // 16, p2_body, 0)
    pltpu.sync_copy(dsich, dsi_hbm.at[pl.ds(db, N_PAD // NSUB)])
    pltpu.sync_copy(dscch, dsc_hbm.at[pl.ds(db, N_PAD // NSUB)])
    plsc.subcore_barrier()

    nbase = c * N2

    def stage(i, p1):
        sn = i >> 2
        qn = sn & 1
        sb = i & 3

        @pl.when(sb == 0)
        def _():
            soff = ebase + sn * SUPER
            pltpu.sync_copy(row_hbm.at[pl.ds(soff, SUPER)], rowsb.at[qn])
            pltpu.sync_copy(col_hbm.at[pl.ds(soff, SUPER)], colsb.at[qn])

        for g in range(GROUPS):
            sl16 = pl.ds(sb * CHUNK + g * 16, 16)
            c16 = colsb[qn, sl16]
            r16 = rowsb[qn, sl16]
            even = iota16 * 2 + g * 32
            p1v = jnp.full((16,), p1, jnp.int32)
            plsc.store_scatter(gidx, [p1v, even], c16 * 2)
            plsc.store_scatter(gidx, [p1v, even + 1], c16 * 2 + 1)
            rcidx[p1, pl.ds(g * 16, 16)] = r16
            rcidx[p1, pl.ds(CHUNK + g * 16, 16)] = c16

        pltpu.async_copy(y_hbm.at[gidx.at[p1]], rows.at[pl.ds(128 * p1, 128), :],
                         semr)
        pltpu.async_copy(dsi_hbm.at[rcidx.at[p1]], dsirc.at[p1], sems)

    stage(jnp.int32(0), jnp.int32(0))

    def weights_chunk(i, p):
        sn = i >> 2
        qn = sn & 1
        sb = i & 3
        pltpu.make_async_copy(dsi_hbm.at[rcidx.at[p]], dsirc.at[p], sems).wait()
        for g in range(GROUPS):
            sl16 = pl.ds(sb * CHUNK + g * 16, 16)
            r16 = rowsb[qn, sl16]
            c16 = colsb[qn, sl16]
            ar, br = _unpack_ab(plsc.load_gather(tab, [r16]))
            ac, bc = _unpack_ab(plsc.load_gather(tab, [c16]))
            m = _tanh16(ar + bc)
            rm = _tanh16(ac + br)
            nd = -(m * rm)
            g16 = pl.ds(g * 16, 16)
            w = dsirc[p, g16] * nd * dsirc[p, pl.ds(CHUNK + g * 16, 16)]
            loc = r16 - nbase
            inb = (loc >= 0) & (loc < N2)
            wb[g16] = jnp.where(inb, w, 0.0)
            loc = jnp.where(inb, loc, 0)
            even = iota16 * 2 + g * 32
            pv = jnp.full((16,), p, jnp.int32)
            plsc.store_scatter(sidx, [pv, even], loc * 2)
            plsc.store_scatter(sidx, [pv, even + 1], loc * 2 + 1)

    def scale_slot(p):
        def scale_body(e2, carry2):
            e = 2 * e2
            ws0 = plsc.load_gather(wb, [jnp.full((16,), e, jnp.int32)])
            ws1 = plsc.load_gather(wb, [jnp.full((16,), e + 1, jnp.int32)])
            r0 = 128 * p + 2 * e
            for j in range(128 // 16):
                sl = pl.ds(j * 16, 16)
                rows[r0, sl] = rows[r0, sl] * ws0
                rows[r0 + 1, sl] = rows[r0 + 1, sl] * ws0
                rows[r0 + 2, sl] = rows[r0 + 2, sl] * ws1
                rows[r0 + 3, sl] = rows[r0 + 3, sl] * ws1
            return carry2

        lax.fori_loop(0, CHUNK // 2, scale_body, 0)

    def p3_pair(i2, carry):
        i = 2 * i2
        stage(i + 1, jnp.int32(1))
        weights_chunk(i, 0)
        pltpu.make_async_copy(y_hbm.at[gidx.at[0]],
                              rows.at[pl.ds(0, 128), :], semr).wait()
        scale_slot(0)
        d0 = pltpu.async_copy(rows.at[pl.ds(0, 128), :], acc.at[sidx.at[0]],
                              semw, add=True)
        weights_chunk(i + 1, 1)
        d0.wait()

        @pl.when(i + 2 < CPT)
        def _():
            stage(i + 2, jnp.int32(0))

        pltpu.make_async_copy(y_hbm.at[gidx.at[1]],
                              rows.at[pl.ds(128, 128), :], semr).wait()
        scale_slot(1)
        pltpu.sync_copy(rows.at[pl.ds(128, 128), :], acc.at[sidx.at[1]],
                        add=True)
        return carry

    lax.fori_loop(0, CPT // 2, p3_pair, 0)
    plsc.subcore_barrier()

    def combine(gb, lb, nr):
        pltpu.sync_copy(x_hbm.at[pl.ds(2 * gb, 2 * nr), :], rows.at[pl.ds(0, 2 * nr), :])
        pltpu.sync_copy(y_hbm.at[pl.ds(2 * gb, 2 * nr), :], rows.at[pl.ds(64, 2 * nr), :])
        pltpu.sync_copy(acc.at[pl.ds(2 * lb, 2 * nr), :], rows.at[pl.ds(128, 2 * nr), :])
        pltpu.sync_copy(dsc_hbm.at[pl.ds(gb, nr)], dscch.at[pl.ds(0, nr)])

        def rbody(lane, carry2):
            dv = plsc.load_gather(dscch, [jnp.full((16,), lane, jnp.int32)])
            for h in range(2):
                r = 2 * lane + h
                for j in range(128 // 16):
                    sl = pl.ds(j * 16, 16)
                    rows[r, sl] = rows[r, sl] - STEP * (
                        dv * rows[64 + r, sl] + rows[128 + r, sl])
            return carry2

        lax.fori_loop(0, nr, rbody, 0)
        pltpu.sync_copy(rows.at[pl.ds(0, 2 * nr), :], out_hbm.at[pl.ds(2 * gb, 2 * nr), :])

    nfull = jnp.where(s == NSUB - 1, (N2 - (NSUB - 1) * TROWS) // PCH,
                      TROWS // PCH)

    def p4_body(k, carry):
        lb = s * TROWS + k * PCH
        combine(nbase + lb, lb, PCH)
        return carry

    lax.fori_loop(0, nfull, p4_body, 0)

    @pl.when(s == NSUB - 1)
    def _():
        lb = (NSUB - 1) * TROWS + ((N2 - (NSUB - 1) * TROWS) // PCH) * PCH
        combine(nbase + lb, lb, NTAIL)


def _sc_stage(x2, y2, abp, row_p, col_p):
    mesh = plsc.VectorSubcoreMesh(core_axis_name="c", subcore_axis_name="s",
                                  num_cores=NCORE, num_subcores=NSUB)
    f = functools.partial(
        pl.kernel,
        mesh=mesh,
        compiler_params=pltpu.CompilerParams(needs_layout_passes=False),
        out_type=[
            jax.ShapeDtypeStruct((2 * N, 128), jnp.float32),
            jax.ShapeDtypeStruct((N_PAD,), jnp.float32),
            jax.ShapeDtypeStruct((N_PAD,), jnp.float32),
        ],
        scratch_types=[
            pltpu.VMEM((N_PAD,), jnp.int32),
            pltpu.VMEM((2, SUPER), jnp.int32),
            pltpu.VMEM((2, SUPER), jnp.int32),
            pltpu.VMEM((256, 128), jnp.float32),
            pltpu.VMEM((2, 2 * CHUNK), jnp.int32),
            pltpu.VMEM((2, 2 * CHUNK), jnp.int32),
            pltpu.VMEM((2, 2 * CHUNK), jnp.int32),
            pltpu.VMEM((2, 2 * CHUNK), jnp.float32),
            pltpu.VMEM((CHUNK,), jnp.float32),
            pltpu.VMEM((2, 2 * CHUNK), jnp.float32),
            pltpu.VMEM((2, 2 * CHUNK), jnp.int32),
            pltpu.VMEM((N_PAD // NSUB,), jnp.float32),
            pltpu.VMEM((N_PAD // NSUB,), jnp.float32),
            pltpu.VMEM((N_PAD // NSUB,), jnp.float32),
            pltpu.VMEM_SHARED((2 * NSUB * TROWS, 128), jnp.float32),
            pltpu.VMEM_SHARED((N_PAD,), jnp.float32),
            pltpu.SemaphoreType.DMA,
            pltpu.SemaphoreType.DMA,
            pltpu.SemaphoreType.DMA,
        ],
    )(_sc_body)
    out2, _, _ = f(x2, y2, abp, row_p, col_p)
    return out2


@jax.jit
def kernel(x, edge_index, W_lin, b_lin, W_sheaf):
    wsp = jnp.zeros((D, 128), jnp.float32)
    wsp = wsp.at[:, 0].set(W_sheaf[:D, 0]).at[:, 1].set(W_sheaf[D:, 0])
    y, abp = _dense(x, W_lin, b_lin.reshape(1, D), wsp)
    a16 = lax.bitcast_convert_type(abp[:, 0].astype(jnp.bfloat16), jnp.uint16)
    b16 = lax.bitcast_convert_type(abp[:, 1].astype(jnp.bfloat16), jnp.uint16)
    packed = (a16.astype(jnp.int32) << 16) | b16.astype(jnp.int32)
    pad = E_PAD - E
    row_p = jnp.concatenate([edge_index[0], jnp.full((pad,), N, jnp.int32)])
    col_p = jnp.concatenate([edge_index[1], jnp.zeros((pad,), jnp.int32)])
    x2 = x.reshape(2 * N, 128)
    y2 = y.reshape(2 * N, 128)
    out2 = _sc_stage(x2, y2, packed, row_p, col_p)
    return out2.reshape(N, D)


# speedup vs baseline: 6.4448x; 1.0337x over previous
import functools

import jax
import jax.numpy as jnp
from jax import lax
from jax.experimental import pallas as pl
from jax.experimental.pallas import tpu as pltpu
import jax.experimental.pallas.tpu_sc as plsc

N = 10000
D = 256
E = 160000
STEP = 0.5

NCORE = 2
NSUB = 16
N2 = N // NCORE
N_PAD = 10240
CHUNK = 64
GROUPS = CHUNK // 16
SUP = 4
SUPER = SUP * CHUNK
SPT = 40
CPT = SPT * SUP
EPT = CPT * CHUNK
E_PAD = EPT * NSUB
TROWS = 320
PCH = 32
NTAIL = 8

MASK_HI = -65536


def _tanh16(z):
    az = jnp.abs(z)
    e = jnp.exp(-2.0 * az)
    p = (1.0 - e) / (1.0 + e)
    return jnp.where(z < 0, -p, p)


def _rsqrt16(u):
    g = plsc.bitcast(jnp.int32(0x5F3759DF) - (plsc.bitcast(u, jnp.int32) >> 1),
                     jnp.float32)
    for _ in range(3):
        g = g * (1.5 - 0.5 * u * g * g)
    return g


def _unpack_ab(v):
    a = plsc.bitcast(v & jnp.int32(MASK_HI), jnp.float32)
    b = plsc.bitcast(v << 16, jnp.float32)
    return a, b


def _dense(x, w_lin, b_lin2, wsp):
    BM = 512
    grid = (pl.cdiv(N, BM),)

    def body(x_ref, wl_ref, bl_ref, ws_ref, y_ref, ab_ref):
        xb = x_ref[...]
        y = lax.dot_general(xb, wl_ref[...], (((1,), (1,)), ((), ())),
                            preferred_element_type=jnp.float32)
        y_ref[...] = y + bl_ref[...]
        ab_ref[...] = jnp.dot(xb, ws_ref[...], preferred_element_type=jnp.float32)

    return pl.pallas_call(
        body,
        grid=grid,
        in_specs=[
            pl.BlockSpec((BM, D), lambda i: (i, 0)),
            pl.BlockSpec((D, D), lambda i: (0, 0)),
            pl.BlockSpec((1, D), lambda i: (0, 0)),
            pl.BlockSpec((D, 128), lambda i: (0, 0)),
        ],
        out_specs=[
            pl.BlockSpec((BM, D), lambda i: (i, 0)),
            pl.BlockSpec((BM, 128), lambda i: (i, 0)),
        ],
        out_shape=[
            jax.ShapeDtypeStruct((N, D), jnp.float32),
            jax.ShapeDtypeStruct((N, 128), jnp.float32),
        ],
    )(x, w_lin, b_lin2, wsp)


def _sc_body(x_hbm, y_hbm, ab_hbm, row_hbm, col_hbm,
             out_hbm, dsi_hbm, dsc_hbm,
             tab, rowsb, colsb, rows, gidx, sidx, rcidx, dsirc,
             wb, m2sb, rowb4,
             dgch, dsich, dscch, acc, dwide, semr, sems, semw, semm):
    c = lax.axis_index("c")
    s = lax.axis_index("s")
    iota16 = lax.iota(jnp.int32, 16)
    zf = jnp.zeros((16,), jnp.float32)
    zi = jnp.zeros((16,), jnp.int32)

    pltpu.sync_copy(ab_hbm, tab.at[pl.ds(0, N)])

    def z_tab(i, carry):
        tab[pl.ds(N + i * 16, 16)] = zi
        return carry

    lax.fori_loop(0, (N_PAD - N) // 16, z_tab, 0)

    def z_rows(r, carry):
        for j in range(128 // 16):
            rows[r, pl.ds(j * 16, 16)] = zf
        return carry

    lax.fori_loop(0, 2 * PCH, z_rows, 0)
    tb = 2 * s * TROWS

    def z_acc(k, carry):
        pltpu.sync_copy(rows.at[pl.ds(0, 2 * PCH), :],
                        acc.at[pl.ds(tb + k * 2 * PCH, 2 * PCH), :])
        return carry

    lax.fori_loop(0, TROWS // PCH, z_acc, 0)
    db = s * (N_PAD // NSUB)

    def z_dsich(i, carry):
        dsich[pl.ds(i * 16, 16)] = zf
        return carry

    lax.fori_loop(0, N_PAD // NSUB // 16, z_dsich, 0)
    pltpu.sync_copy(dsich, dwide.at[pl.ds(db, N_PAD // NSUB)])
    plsc.subcore_barrier()

    ebase = s * EPT

    def p1_fire_meta(s2, q):
        soff = ebase + s2 * SUPER
        pltpu.async_copy(row_hbm.at[pl.ds(soff, SUPER)], rowsb.at[q], sems)
        pltpu.async_copy(col_hbm.at[pl.ds(soff, SUPER)], colsb.at[q], sems)

    p1_fire_meta(jnp.int32(0), jnp.int32(0))

    def p1_super(s2, carry):
        q = s2 & 1
        pltpu.make_async_copy(row_hbm.at[pl.ds(0, SUPER)], rowsb.at[q],
                              sems).wait()
        pltpu.make_async_copy(col_hbm.at[pl.ds(0, SUPER)], colsb.at[q],
                              sems).wait()

        @pl.when(s2 + 1 < SPT)
        def _():
            p1_fire_meta(s2 + 1, 1 - q)

        def p1_sub(h, carry2):
            for g in range(2 * GROUPS):
                sl16 = pl.ds(h * 2 * CHUNK + g * 16, 16)
                r16 = rowsb[q, sl16]
                c16 = colsb[q, sl16]
                ar, br = _unpack_ab(plsc.load_gather(tab, [r16]))
                ac, bc = _unpack_ab(plsc.load_gather(tab, [c16]))
                m = _tanh16(ar + bc)
                m2sb[h, pl.ds(g * 16, 16)] = m * m
                rowb4[h, pl.ds(g * 16, 16)] = r16
            pltpu.sync_copy(m2sb.at[h], dwide.at[rowb4.at[h]], add=True)
            return carry2

        lax.fori_loop(0, SUP // 2, p1_sub, 0)
        return carry

    lax.fori_loop(0, SPT, p1_super, 0)
    plsc.subcore_barrier()

    pltpu.sync_copy(dwide.at[pl.ds(db, N_PAD // NSUB)], dgch)

    def p2_body(i, carry):
        v = dgch[pl.ds(i * 16, 16)]
        u = v + 1.0
        dsich[pl.ds(i * 16, 16)] = _rsqrt16(u)
        dscch[pl.ds(i * 16, 16)] = v / u
        return carry

    lax.fori_loop(0, N_PAD // NSUB // 16, p2_body, 0)
    pltpu.sync_copy(dsich, dsi_hbm.at[pl.ds(db, N_PAD // NSUB)])
    pltpu.sync_copy(dscch, dsc_hbm.at[pl.ds(db, N_PAD // NSUB)])
    plsc.subcore_barrier()

    nbase = c * N2

    def fire_meta(sn, q):
        soff = ebase + sn * SUPER
        pltpu.async_copy(row_hbm.at[pl.ds(soff, SUPER)], rowsb.at[q], semm)
        pltpu.async_copy(col_hbm.at[pl.ds(soff, SUPER)], colsb.at[q], semm)

    fire_meta(jnp.int32(0), jnp.int32(0))

    def stage(i, p1):
        sn = i >> 2
        qn = sn & 1
        sb = i & 3

        @pl.when(sb == 0)
        def _():
            pltpu.make_async_copy(row_hbm.at[pl.ds(0, SUPER)], rowsb.at[qn],
                                  semm).wait()
            pltpu.make_async_copy(col_hbm.at[pl.ds(0, SUPER)], colsb.at[qn],
                                  semm).wait()

            @pl.when(sn + 1 < SPT)
            def _():
                fire_meta(sn + 1, 1 - qn)

        for g in range(GROUPS):
            sl16 = pl.ds(sb * CHUNK + g * 16, 16)
            c16 = colsb[qn, sl16]
            r16 = rowsb[qn, sl16]
            even = iota16 * 2 + g * 32
            p1v = jnp.full((16,), p1, jnp.int32)
            plsc.store_scatter(gidx, [p1v, even], c16 * 2)
            plsc.store_scatter(gidx, [p1v, even + 1], c16 * 2 + 1)
            rcidx[p1, pl.ds(g * 16, 16)] = r16
            rcidx[p1, pl.ds(CHUNK + g * 16, 16)] = c16

        pltpu.async_copy(y_hbm.at[gidx.at[p1]], rows.at[pl.ds(128 * p1, 128), :],
                         semr)
        pltpu.async_copy(dsi_hbm.at[rcidx.at[p1]], dsirc.at[p1], sems)

    stage(jnp.int32(0), jnp.int32(0))

    def weights_chunk(i, p):
        sn = i >> 2
        qn = sn & 1
        sb = i & 3
        pltpu.make_async_copy(dsi_hbm.at[rcidx.at[p]], dsirc.at[p], sems).wait()
        for g in range(GROUPS):
            sl16 = pl.ds(sb * CHUNK + g * 16, 16)
            r16 = rowsb[qn, sl16]
            c16 = colsb[qn, sl16]
            ar, br = _unpack_ab(plsc.load_gather(tab, [r16]))
            ac, bc = _unpack_ab(plsc.load_gather(tab, [c16]))
            m = _tanh16(ar + bc)
            rm = _tanh16(ac + br)
            nd = -(m * rm)
            g16 = pl.ds(g * 16, 16)
            w = dsirc[p, g16] * nd * dsirc[p, pl.ds(CHUNK + g * 16, 16)]
            loc = r16 - nbase
            inb = (loc >= 0) & (loc < N2)
            wb[g16] = jnp.where(inb, w, 0.0)
            loc = jnp.where(inb, loc, 0)
            even = iota16 * 2 + g * 32
            pv = jnp.full((16,), p, jnp.int32)
            plsc.store_scatter(sidx, [pv, even], loc * 2)
            plsc.store_scatter(sidx, [pv, even + 1], loc * 2 + 1)

    def scale_slot(p):
        def scale_body(e2, carry2):
            e = 2 * e2
            ws0 = plsc.load_gather(wb, [jnp.full((16,), e, jnp.int32)])
            ws1 = plsc.load_gather(wb, [jnp.full((16,), e + 1, jnp.int32)])
            r0 = 128 * p + 2 * e
            for j in range(128 // 16):
                sl = pl.ds(j * 16, 16)
                rows[r0, sl] = rows[r0, sl] * ws0
                rows[r0 + 1, sl] = rows[r0 + 1, sl] * ws0
                rows[r0 + 2, sl] = rows[r0 + 2, sl] * ws1
                rows[r0 + 3, sl] = rows[r0 + 3, sl] * ws1
            return carry2

        lax.fori_loop(0, CHUNK // 2, scale_body, 0)

    def p3_pair(i2, carry):
        i = 2 * i2
        stage(i + 1, jnp.int32(1))
        weights_chunk(i, 0)
        pltpu.make_async_copy(y_hbm.at[gidx.at[0]],
                              rows.at[pl.ds(0, 128), :], semr).wait()
        scale_slot(0)
        d0 = pltpu.async_copy(rows.at[pl.ds(0, 128), :], acc.at[sidx.at[0]],
                              semw, add=True)
        weights_chunk(i + 1, 1)
        d0.wait()

        @pl.when(i + 2 < CPT)
        def _():
            stage(i + 2, jnp.int32(0))

        pltpu.make_async_copy(y_hbm.at[gidx.at[1]],
                              rows.at[pl.ds(128, 128), :], semr).wait()
        scale_slot(1)
        pltpu.sync_copy(rows.at[pl.ds(128, 128), :], acc.at[sidx.at[1]],
                        add=True)
        return carry

    lax.fori_loop(0, CPT // 2, p3_pair, 0)
    plsc.subcore_barrier()

    def combine(gb, lb, nr):
        pltpu.sync_copy(x_hbm.at[pl.ds(2 * gb, 2 * nr), :], rows.at[pl.ds(0, 2 * nr), :])
        pltpu.sync_copy(y_hbm.at[pl.ds(2 * gb, 2 * nr), :], rows.at[pl.ds(64, 2 * nr), :])
        pltpu.sync_copy(acc.at[pl.ds(2 * lb, 2 * nr), :], rows.at[pl.ds(128, 2 * nr), :])
        pltpu.sync_copy(dsc_hbm.at[pl.ds(gb, nr)], dscch.at[pl.ds(0, nr)])

        def rbody(lane, carry2):
            dv = plsc.load_gather(dscch, [jnp.full((16,), lane, jnp.int32)])
            for h in range(2):
                r = 2 * lane + h
                for j in range(128 // 16):
                    sl = pl.ds(j * 16, 16)
                    rows[r, sl] = rows[r, sl] - STEP * (
                        dv * rows[64 + r, sl] + rows[128 + r, sl])
            return carry2

        lax.fori_loop(0, nr, rbody, 0)
        pltpu.sync_copy(rows.at[pl.ds(0, 2 * nr), :], out_hbm.at[pl.ds(2 * gb, 2 * nr), :])

    nfull = jnp.where(s == NSUB - 1, (N2 - (NSUB - 1) * TROWS) // PCH,
                      TROWS // PCH)

    def p4_body(k, carry):
        lb = s * TROWS + k * PCH
        combine(nbase + lb, lb, PCH)
        return carry

    lax.fori_loop(0, nfull, p4_body, 0)

    @pl.when(s == NSUB - 1)
    def _():
        lb = (NSUB - 1) * TROWS + ((N2 - (NSUB - 1) * TROWS) // PCH) * PCH
        combine(nbase + lb, lb, NTAIL)


def _sc_stage(x2, y2, abp, row_p, col_p):
    mesh = plsc.VectorSubcoreMesh(core_axis_name="c", subcore_axis_name="s",
                                  num_cores=NCORE, num_subcores=NSUB)
    f = functools.partial(
        pl.kernel,
        mesh=mesh,
        compiler_params=pltpu.CompilerParams(needs_layout_passes=False),
        out_type=[
            jax.ShapeDtypeStruct((2 * N, 128), jnp.float32),
            jax.ShapeDtypeStruct((N_PAD,), jnp.float32),
            jax.ShapeDtypeStruct((N_PAD,), jnp.float32),
        ],
        scratch_types=[
            pltpu.VMEM((N_PAD,), jnp.int32),
            pltpu.VMEM((2, SUPER), jnp.int32),
            pltpu.VMEM((2, SUPER), jnp.int32),
            pltpu.VMEM((256, 128), jnp.float32),
            pltpu.VMEM((2, 2 * CHUNK), jnp.int32),
            pltpu.VMEM((2, 2 * CHUNK), jnp.int32),
            pltpu.VMEM((2, 2 * CHUNK), jnp.int32),
            pltpu.VMEM((2, 2 * CHUNK), jnp.float32),
            pltpu.VMEM((CHUNK,), jnp.float32),
            pltpu.VMEM((2, 2 * CHUNK), jnp.float32),
            pltpu.VMEM((2, 2 * CHUNK), jnp.int32),
            pltpu.VMEM((N_PAD // NSUB,), jnp.float32),
            pltpu.VMEM((N_PAD // NSUB,), jnp.float32),
            pltpu.VMEM((N_PAD // NSUB,), jnp.float32),
            pltpu.VMEM_SHARED((2 * NSUB * TROWS, 128), jnp.float32),
            pltpu.VMEM_SHARED((N_PAD,), jnp.float32),
            pltpu.SemaphoreType.DMA,
            pltpu.SemaphoreType.DMA,
            pltpu.SemaphoreType.DMA,
            pltpu.SemaphoreType.DMA,
        ],
    )(_sc_body)
    out2, _, _ = f(x2, y2, abp, row_p, col_p)
    return out2


@jax.jit
def kernel(x, edge_index, W_lin, b_lin, W_sheaf):
    wsp = jnp.zeros((D, 128), jnp.float32)
    wsp = wsp.at[:, 0].set(W_sheaf[:D, 0]).at[:, 1].set(W_sheaf[D:, 0])
    y, abp = _dense(x, W_lin, b_lin.reshape(1, D), wsp)
    a16 = lax.bitcast_convert_type(abp[:, 0].astype(jnp.bfloat16), jnp.uint16)
    b16 = lax.bitcast_convert_type(abp[:, 1].astype(jnp.bfloat16), jnp.uint16)
    packed = (a16.astype(jnp.int32) << 16) | b16.astype(jnp.int32)
    pad = E_PAD - E
    row_p = jnp.concatenate([edge_index[0], jnp.full((pad,), N, jnp.int32)])
    col_p = jnp.concatenate([edge_index[1], jnp.zeros((pad,), jnp.int32)])
    x2 = x.reshape(2 * N, 128)
    y2 = y.reshape(2 * N, 128)
    out2 = _sc_stage(x2, y2, packed, row_p, col_p)
    return out2.reshape(N, D)
